# Initial kernel scaffold; baseline (speedup 1.0000x reference)
import jax
import jax.numpy as jnp
from jax.experimental import pallas as pl


def kernel(x, edge_index, W1, b1, W2, b2, W3, b3, We, be, Wh, bh, Wg, bg):
    raise NotImplementedError("write your pallas kernel here")


# speedup vs baseline: 12.0289x; 12.0289x over previous
import dataclasses
import functools

import jax
import jax.numpy as jnp
from jax import lax
from jax.experimental import pallas as pl
from jax.experimental.pallas import tpu as pltpu
from jax.experimental.pallas import tpu_sc as plsc

N = 10000
NP = 10240
E = 320000
EP = 327680
W_EDGE = 128
WINDOWS = EP // W_EDGE
N_TILES = 32
PW = WINDOWS // N_TILES
ROWS_PER_TILE = NP // 16

_MESH = plsc.VectorSubcoreMesh(core_axis_name="c", subcore_axis_name="s")

_CP = pltpu.CompilerParams(needs_layout_passes=False)
_CP_UNTILED = pltpu.CompilerParams(use_tc_tiling_on_sc=False)


@functools.partial(
    pl.kernel,
    out_type=jax.ShapeDtypeStruct((N_TILES, NP // 128, 128), jnp.float32),
    mesh=_MESH,
    compiler_params=_CP,
    scratch_types=[
        pltpu.VMEM((PW, W_EDGE), jnp.int32),
        pltpu.VMEM((NP // 128, 128), jnp.float32),
    ],
)
def _sc_degree(dst_hbm, out_hbm, idx_v, hist_v):
    c = lax.axis_index("c")
    s = lax.axis_index("s")
    wid = c * 16 + s
    zero16 = jnp.zeros((16,), jnp.float32)
    one16 = jnp.ones((16,), jnp.float32)

    @pl.loop(0, NP // 128)
    def _(r):
        @pl.loop(0, 128, step=16)
        def _(j):
            hist_v[r, pl.ds(j, 16)] = zero16

    pltpu.sync_copy(dst_hbm.at[pl.ds(wid * PW, PW)], idx_v)

    @pl.loop(0, PW)
    def _(w):
        @pl.loop(0, W_EDGE, step=16)
        def _(j):
            v = idx_v[w, pl.ds(j, 16)]
            plsc.addupdate_scatter(hist_v, [v >> 7, v & 127], one16)

    pltpu.sync_copy(hist_v, out_hbm.at[wid])


def _make_sc_scatter(F):
    @functools.partial(
        pl.kernel,
        out_type=jax.ShapeDtypeStruct((2, NP, F), jnp.float32),
        mesh=_MESH,
        compiler_params=_CP_UNTILED,
        scratch_types=[
            pltpu.VMEM((PW, W_EDGE), jnp.int32),
            pltpu.VMEM((PW, W_EDGE), jnp.int32),
            pltpu.VMEM((W_EDGE, F), jnp.float32),
            pltpu.VMEM((W_EDGE, F), jnp.float32),
            pltpu.VMEM_SHARED((NP, F), jnp.float32),
        ],
    )
    def k(g_hbm, src_hbm, dst_hbm, out_hbm, srcv, dstv, rows, zbuf, acc_sh):
        c = lax.axis_index("c")
        s = lax.axis_index("s")
        wid = c * 16 + s
        zero16 = jnp.zeros((16,), jnp.float32)

        @pl.loop(0, W_EDGE)
        def _(r):
            @pl.loop(0, F, step=16)
            def _(j):
                zbuf[r, pl.ds(j, 16)] = zero16

        @pl.loop(0, ROWS_PER_TILE, step=W_EDGE)
        def _(kk):
            pltpu.sync_copy(zbuf, acc_sh.at[pl.ds(s * ROWS_PER_TILE + kk, W_EDGE)])

        plsc.subcore_barrier()

        pltpu.sync_copy(src_hbm.at[pl.ds(wid * PW, PW)], srcv)
        pltpu.sync_copy(dst_hbm.at[pl.ds(wid * PW, PW)], dstv)

        @pl.loop(0, PW)
        def _(w):
            pltpu.sync_copy(g_hbm.at[srcv.at[w]], rows)
            pltpu.sync_copy(rows, acc_sh.at[dstv.at[w]], add=True)

        plsc.subcore_barrier()

        pltpu.sync_copy(
            acc_sh.at[pl.ds(s * ROWS_PER_TILE, ROWS_PER_TILE)],
            out_hbm.at[c, pl.ds(s * ROWS_PER_TILE, ROWS_PER_TILE)],
        )

    return k


_sc_scatter64 = _make_sc_scatter(64)
_sc_scatter32 = _make_sc_scatter(32)


_BN = 640


def _tc_layer0(x_p, W1, deg_parts):
    def body(deg_ref, x_ref, w_ref, g_ref, dinv_ref):
        degT = jnp.transpose(deg_ref[...])
        deg = jnp.sum(degT, axis=1, keepdims=True) + 1.0
        dinv = lax.rsqrt(deg)
        xs = x_ref[...] * dinv
        g_ref[...] = jnp.dot(xs, w_ref[...], preferred_element_type=jnp.float32)
        dinv_ref[...] = dinv

    return pl.pallas_call(
        body,
        grid=(NP // _BN,),
        in_specs=[
            pl.BlockSpec((N_TILES, _BN), lambda i: (0, i)),
            pl.BlockSpec((_BN, 128), lambda i: (i, 0)),
            pl.BlockSpec((128, 64), lambda i: (0, 0)),
        ],
        out_specs=[
            pl.BlockSpec((_BN, 64), lambda i: (i, 0)),
            pl.BlockSpec((_BN, 1), lambda i: (i, 0)),
        ],
        out_shape=[
            jax.ShapeDtypeStruct((NP, 64), jnp.float32),
            jax.ShapeDtypeStruct((NP, 1), jnp.float32),
        ],
    )(deg_parts, x_p, W1)


def _tc_layer(parts, g, dinv, b, Wn, F, Fn):
    def body(p_ref, g_ref, dinv_ref, b_ref, w_ref, o_ref):
        p = p_ref[...]
        t = (p[0] + p[1] + g_ref[...]) * dinv_ref[...]
        h = jnp.maximum(t + b_ref[...], 0.0)
        o_ref[...] = jnp.dot(h * dinv_ref[...], w_ref[...],
                             preferred_element_type=jnp.float32)

    return pl.pallas_call(
        body,
        grid=(NP // _BN,),
        in_specs=[
            pl.BlockSpec((2, _BN, F), lambda i: (0, i, 0)),
            pl.BlockSpec((_BN, F), lambda i: (i, 0)),
            pl.BlockSpec((_BN, 1), lambda i: (i, 0)),
            pl.BlockSpec((1, F), lambda i: (0, 0)),
            pl.BlockSpec((F, Fn), lambda i: (0, 0)),
        ],
        out_specs=pl.BlockSpec((_BN, Fn), lambda i: (i, 0)),
        out_shape=jax.ShapeDtypeStruct((NP, Fn), jnp.float32),
    )(parts, g, dinv, b.reshape(1, F), Wn)


def _tc_final(parts, g, dinv, b, Wcat, bcat):
    def body(p_ref, g_ref, dinv_ref, b_ref, w_ref, bc_ref, o_ref):
        p = p_ref[...]
        t = (p[0] + p[1] + g_ref[...]) * dinv_ref[...]
        h = jnp.maximum(t + b_ref[...], 0.0)
        o_ref[...] = jnp.dot(h, w_ref[...],
                             preferred_element_type=jnp.float32) + bc_ref[...]

    return pl.pallas_call(
        body,
        grid=(NP // _BN,),
        in_specs=[
            pl.BlockSpec((2, _BN, 32), lambda i: (0, i, 0)),
            pl.BlockSpec((_BN, 32), lambda i: (i, 0)),
            pl.BlockSpec((_BN, 1), lambda i: (i, 0)),
            pl.BlockSpec((1, 32), lambda i: (0, 0)),
            pl.BlockSpec((32, 9), lambda i: (0, 0)),
            pl.BlockSpec((1, 9), lambda i: (0, 0)),
        ],
        out_specs=pl.BlockSpec((_BN, 9), lambda i: (i, 0)),
        out_shape=jax.ShapeDtypeStruct((NP, 9), jnp.float32),
    )(parts, g, dinv, b.reshape(1, 32), Wcat, bcat.reshape(1, 9))


def kernel(x, edge_index, W1, b1, W2, b2, W3, b3, We, be, Wh, bh, Wg, bg):
    src = edge_index[0]
    dst = edge_index[1]
    pad = jnp.full((EP - E,), N, jnp.int32)
    srcp = jnp.concatenate([src, pad]).reshape(WINDOWS, W_EDGE)
    dstp = jnp.concatenate([dst, pad]).reshape(WINDOWS, W_EDGE)
    x_p = jnp.pad(x, ((0, NP - N), (0, 0)))

    deg_parts = _sc_degree(dstp).reshape(N_TILES, NP)
    g1, dinv = _tc_layer0(x_p, W1, deg_parts)
    p1 = _sc_scatter64(g1, srcp, dstp)
    g2 = _tc_layer(p1, g1, dinv, b1, W2, 64, 64)
    p2 = _sc_scatter64(g2, srcp, dstp)
    g3 = _tc_layer(p2, g2, dinv, b2, W3, 64, 32)
    p3 = _sc_scatter32(g3, srcp, dstp)

    Wcat = jnp.concatenate([We, Wh, Wg], axis=1)
    bcat = jnp.concatenate([be, bh, bg])
    out = _tc_final(p3, g3, dinv, b3, Wcat, bcat)
    return out[:N]


# speedup vs baseline: 38.7558x; 3.2219x over previous
import dataclasses
import functools

import jax
import jax.numpy as jnp
from jax import lax
from jax.experimental import pallas as pl
from jax.experimental.pallas import tpu as pltpu
from jax.experimental.pallas import tpu_sc as plsc

N = 10000
NP = 10240
E = 320000
EP = 327680
W_EDGE = 128
WINDOWS = EP // W_EDGE
N_TILES = 32
PW = WINDOWS // N_TILES
ROWS_PER_TILE = NP // 16

_MESH = plsc.VectorSubcoreMesh(core_axis_name="c", subcore_axis_name="s")

_CP = pltpu.CompilerParams(needs_layout_passes=False)
_CP_UNTILED = pltpu.CompilerParams(use_tc_tiling_on_sc=False)


@functools.partial(
    pl.kernel,
    out_type=jax.ShapeDtypeStruct((N_TILES, NP // 128, 128), jnp.float32),
    mesh=_MESH,
    compiler_params=_CP,
    scratch_types=[
        pltpu.VMEM((PW, W_EDGE), jnp.int32),
        pltpu.VMEM((NP // 128, 128), jnp.float32),
    ],
)
def _sc_degree(dst_hbm, out_hbm, idx_v, hist_v):
    c = lax.axis_index("c")
    s = lax.axis_index("s")
    wid = c * 16 + s
    zero16 = jnp.zeros((16,), jnp.float32)
    one16 = jnp.ones((16,), jnp.float32)

    @pl.loop(0, NP // 128)
    def _(r):
        @pl.loop(0, 128, step=16)
        def _(j):
            hist_v[r, pl.ds(j, 16)] = zero16

    pltpu.sync_copy(dst_hbm.at[pl.ds(wid * PW, PW)], idx_v)

    @pl.loop(0, PW)
    def _(w):
        @pl.loop(0, W_EDGE, step=16)
        def _(j):
            v = idx_v[w, pl.ds(j, 16)]
            plsc.addupdate_scatter(hist_v, [v >> 7, v & 127], one16)

    pltpu.sync_copy(hist_v, out_hbm.at[wid])


def _make_sc_scatter(F):
    NBUF = 4

    @functools.partial(
        pl.kernel,
        out_type=jax.ShapeDtypeStruct((2, NP, F), jnp.float32),
        mesh=_MESH,
        compiler_params=_CP_UNTILED,
        scratch_types=[
            pltpu.VMEM((PW, W_EDGE), jnp.int32),
            pltpu.VMEM((PW, W_EDGE), jnp.int32),
            pltpu.VMEM((NBUF, W_EDGE, F), jnp.float32),
            pltpu.VMEM((W_EDGE, F), jnp.float32),
            pltpu.VMEM_SHARED((NP, F), jnp.float32),
        ]
        + [pltpu.SemaphoreType.DMA] * (2 * NBUF),
    )
    def k(g_hbm, src_hbm, dst_hbm, out_hbm, srcv, dstv, rows, zbuf, acc_sh, *sems):
        gsem = sems[:NBUF]
        ssem = sems[NBUF:]
        c = lax.axis_index("c")
        s = lax.axis_index("s")
        wid = c * 16 + s
        zero16 = jnp.zeros((16,), jnp.float32)

        @pl.loop(0, W_EDGE)
        def _(r):
            @pl.loop(0, F, step=16)
            def _(j):
                zbuf[r, pl.ds(j, 16)] = zero16

        @pl.loop(0, ROWS_PER_TILE, step=W_EDGE)
        def _(kk):
            pltpu.sync_copy(zbuf, acc_sh.at[pl.ds(s * ROWS_PER_TILE + kk, W_EDGE)])

        plsc.subcore_barrier()

        pltpu.sync_copy(src_hbm.at[pl.ds(wid * PW, PW)], srcv)
        pltpu.sync_copy(dst_hbm.at[pl.ds(wid * PW, PW)], dstv)

        def sg(w, i):
            pltpu.async_copy(g_hbm.at[srcv.at[w]], rows.at[i], gsem[i])

        def wg(i):
            pltpu.make_async_copy(g_hbm.at[srcv.at[0]], rows.at[i], gsem[i]).wait()

        def ss(w, i):
            pltpu.async_copy(rows.at[i], acc_sh.at[dstv.at[w]], ssem[i], add=True)

        def ws(i):
            pltpu.make_async_copy(rows.at[i], acc_sh.at[dstv.at[0]], ssem[i]).wait()

        for i in range(NBUF):
            sg(i, i)
        for i in range(NBUF):
            wg(i)
            ss(i, i)

        @pl.loop(NBUF, PW, step=NBUF)
        def _(w):
            for i in range(NBUF):
                ws(i)
                sg(w + i, i)
            for i in range(NBUF):
                wg(i)
                ss(w + i, i)

        for i in range(NBUF):
            ws(i)

        plsc.subcore_barrier()

        pltpu.sync_copy(
            acc_sh.at[pl.ds(s * ROWS_PER_TILE, ROWS_PER_TILE)],
            out_hbm.at[c, pl.ds(s * ROWS_PER_TILE, ROWS_PER_TILE)],
        )

    return k


_sc_scatter64 = _make_sc_scatter(64)
_sc_scatter32 = _make_sc_scatter(32)


_BN = 640


def _tc_layer0(x_p, W1, deg_parts):
    def body(deg_ref, x_ref, w_ref, g_ref, dinv_ref):
        degT = jnp.transpose(deg_ref[...])
        deg = jnp.sum(degT, axis=1, keepdims=True) + 1.0
        dinv = lax.rsqrt(deg)
        xs = x_ref[...] * dinv
        g_ref[...] = jnp.dot(xs, w_ref[...], preferred_element_type=jnp.float32)
        dinv_ref[...] = dinv

    return pl.pallas_call(
        body,
        grid=(NP // _BN,),
        in_specs=[
            pl.BlockSpec((N_TILES, _BN), lambda i: (0, i)),
            pl.BlockSpec((_BN, 128), lambda i: (i, 0)),
            pl.BlockSpec((128, 64), lambda i: (0, 0)),
        ],
        out_specs=[
            pl.BlockSpec((_BN, 64), lambda i: (i, 0)),
            pl.BlockSpec((_BN, 1), lambda i: (i, 0)),
        ],
        out_shape=[
            jax.ShapeDtypeStruct((NP, 64), jnp.float32),
            jax.ShapeDtypeStruct((NP, 1), jnp.float32),
        ],
    )(deg_parts, x_p, W1)


def _tc_layer(parts, g, dinv, b, Wn, F, Fn):
    def body(p_ref, g_ref, dinv_ref, b_ref, w_ref, o_ref):
        p = p_ref[...]
        t = (p[0] + p[1] + g_ref[...]) * dinv_ref[...]
        h = jnp.maximum(t + b_ref[...], 0.0)
        o_ref[...] = jnp.dot(h * dinv_ref[...], w_ref[...],
                             preferred_element_type=jnp.float32)

    return pl.pallas_call(
        body,
        grid=(NP // _BN,),
        in_specs=[
            pl.BlockSpec((2, _BN, F), lambda i: (0, i, 0)),
            pl.BlockSpec((_BN, F), lambda i: (i, 0)),
            pl.BlockSpec((_BN, 1), lambda i: (i, 0)),
            pl.BlockSpec((1, F), lambda i: (0, 0)),
            pl.BlockSpec((F, Fn), lambda i: (0, 0)),
        ],
        out_specs=pl.BlockSpec((_BN, Fn), lambda i: (i, 0)),
        out_shape=jax.ShapeDtypeStruct((NP, Fn), jnp.float32),
    )(parts, g, dinv, b.reshape(1, F), Wn)


def _tc_final(parts, g, dinv, b, Wcat, bcat):
    def body(p_ref, g_ref, dinv_ref, b_ref, w_ref, bc_ref, o_ref):
        p = p_ref[...]
        t = (p[0] + p[1] + g_ref[...]) * dinv_ref[...]
        h = jnp.maximum(t + b_ref[...], 0.0)
        o_ref[...] = jnp.dot(h, w_ref[...],
                             preferred_element_type=jnp.float32) + bc_ref[...]

    return pl.pallas_call(
        body,
        grid=(NP // _BN,),
        in_specs=[
            pl.BlockSpec((2, _BN, 32), lambda i: (0, i, 0)),
            pl.BlockSpec((_BN, 32), lambda i: (i, 0)),
            pl.BlockSpec((_BN, 1), lambda i: (i, 0)),
            pl.BlockSpec((1, 32), lambda i: (0, 0)),
            pl.BlockSpec((32, 9), lambda i: (0, 0)),
            pl.BlockSpec((1, 9), lambda i: (0, 0)),
        ],
        out_specs=pl.BlockSpec((_BN, 9), lambda i: (i, 0)),
        out_shape=jax.ShapeDtypeStruct((NP, 9), jnp.float32),
    )(parts, g, dinv, b.reshape(1, 32), Wcat, bcat.reshape(1, 9))


def kernel(x, edge_index, W1, b1, W2, b2, W3, b3, We, be, Wh, bh, Wg, bg):
    src = edge_index[0]
    dst = edge_index[1]
    pad = N + jnp.arange(EP - E, dtype=jnp.int32) % (NP - N)
    srcp = jnp.concatenate([src, pad]).reshape(WINDOWS, W_EDGE)
    dstp = jnp.concatenate([dst, pad]).reshape(WINDOWS, W_EDGE)
    x_p = jnp.pad(x, ((0, NP - N), (0, 0)))

    deg_parts = _sc_degree(dstp).reshape(N_TILES, NP)
    g1, dinv = _tc_layer0(x_p, W1, deg_parts)
    p1 = _sc_scatter64(g1, srcp, dstp)
    g2 = _tc_layer(p1, g1, dinv, b1, W2, 64, 64)
    p2 = _sc_scatter64(g2, srcp, dstp)
    g3 = _tc_layer(p2, g2, dinv, b2, W3, 64, 32)
    p3 = _sc_scatter32(g3, srcp, dstp)

    Wcat = jnp.concatenate([We, Wh, Wg], axis=1)
    bcat = jnp.concatenate([be, bh, bg])
    out = _tc_final(p3, g3, dinv, b3, Wcat, bcat)
    return out[:N]


# speedup vs baseline: 40.8948x; 1.0552x over previous
import functools

import numpy as np

import jax
import jax.numpy as jnp
from jax import lax
from jax.experimental import pallas as pl
from jax.experimental.pallas import tpu as pltpu
from jax.experimental.pallas import tpu_sc as plsc

N = 10000
NP = 10240
E = 320000
W_EDGE = 128
RW = E // W_EDGE
PW = 80
PADW = 32 * PW - RW
N_TILES = 32
ROWS_PER_TILE = NP // 16

_PAD_IDX = (
    N + (np.arange(PADW * W_EDGE, dtype=np.int32) % (NP - N)).reshape(PADW, W_EDGE)
)

_MESH = plsc.VectorSubcoreMesh(core_axis_name="c", subcore_axis_name="s")

_CP = pltpu.CompilerParams(needs_layout_passes=False)
_CP_UNTILED = pltpu.CompilerParams(use_tc_tiling_on_sc=False)


def _load_windows(hbm, pad_hbm, dest, wid):
    @pl.when(wid < N_TILES - 1)
    def _():
        pltpu.sync_copy(hbm.at[pl.ds(wid * PW, PW)], dest)

    @pl.when(wid == N_TILES - 1)
    def _():
        pltpu.sync_copy(hbm.at[pl.ds(RW - (PW - PADW), PW - PADW)],
                        dest.at[pl.ds(0, PW - PADW)])
        pltpu.sync_copy(pad_hbm, dest.at[pl.ds(PW - PADW, PADW)])


@functools.partial(
    pl.kernel,
    out_type=jax.ShapeDtypeStruct((N_TILES, NP // 128, 128), jnp.float32),
    mesh=_MESH,
    compiler_params=_CP,
    scratch_types=[
        pltpu.VMEM((PW, W_EDGE), jnp.int32),
        pltpu.VMEM((NP // 128, 128), jnp.float32),
    ],
)
def _sc_degree(dst_hbm, pad_hbm, out_hbm, idx_v, hist_v):
    c = lax.axis_index("c")
    s = lax.axis_index("s")
    wid = c * 16 + s
    zero16 = jnp.zeros((16,), jnp.float32)
    one16 = jnp.ones((16,), jnp.float32)

    @pl.loop(0, NP // 128)
    def _(r):
        @pl.loop(0, 128, step=16)
        def _(j):
            hist_v[r, pl.ds(j, 16)] = zero16

    _load_windows(dst_hbm, pad_hbm, idx_v, wid)

    @pl.loop(0, PW)
    def _(w):
        @pl.loop(0, W_EDGE, step=16)
        def _(j):
            v = idx_v[w, pl.ds(j, 16)]
            plsc.addupdate_scatter(hist_v, [v >> 7, v & 127], one16)

    pltpu.sync_copy(hist_v, out_hbm.at[wid])


def _make_sc_scatter(F):
    NBUF = 4

    @functools.partial(
        pl.kernel,
        out_type=jax.ShapeDtypeStruct((2, NP, F), jnp.float32),
        mesh=_MESH,
        compiler_params=_CP_UNTILED,
        scratch_types=[
            pltpu.VMEM((PW, W_EDGE), jnp.int32),
            pltpu.VMEM((PW, W_EDGE), jnp.int32),
            pltpu.VMEM((NBUF, W_EDGE, F), jnp.float32),
            pltpu.VMEM((W_EDGE, F), jnp.float32),
            pltpu.VMEM_SHARED((NP, F), jnp.float32),
        ]
        + [pltpu.SemaphoreType.DMA] * (2 * NBUF),
    )
    def k(g_hbm, src_hbm, dst_hbm, pad_hbm, out_hbm, srcv, dstv, rows, zbuf,
          acc_sh, *sems):
        gsem = sems[:NBUF]
        ssem = sems[NBUF:]
        c = lax.axis_index("c")
        s = lax.axis_index("s")
        wid = c * 16 + s
        zero16 = jnp.zeros((16,), jnp.float32)

        @pl.loop(0, W_EDGE)
        def _(r):
            @pl.loop(0, F, step=16)
            def _(j):
                zbuf[r, pl.ds(j, 16)] = zero16

        @pl.loop(0, ROWS_PER_TILE, step=W_EDGE)
        def _(kk):
            pltpu.sync_copy(zbuf, acc_sh.at[pl.ds(s * ROWS_PER_TILE + kk, W_EDGE)])

        plsc.subcore_barrier()

        _load_windows(src_hbm, pad_hbm, srcv, wid)
        _load_windows(dst_hbm, pad_hbm, dstv, wid)

        def sg(w, i):
            pltpu.async_copy(g_hbm.at[srcv.at[w]], rows.at[i], gsem[i])

        def wg(i):
            pltpu.make_async_copy(g_hbm.at[srcv.at[0]], rows.at[i], gsem[i]).wait()

        def ss(w, i):
            pltpu.async_copy(rows.at[i], acc_sh.at[dstv.at[w]], ssem[i], add=True)

        def ws(i):
            pltpu.make_async_copy(rows.at[i], acc_sh.at[dstv.at[0]], ssem[i]).wait()

        for i in range(NBUF):
            sg(i, i)
        for i in range(NBUF):
            wg(i)
            ss(i, i)

        @pl.loop(NBUF, PW, step=NBUF)
        def _(w):
            for i in range(NBUF):
                ws(i)
                sg(w + i, i)
            for i in range(NBUF):
                wg(i)
                ss(w + i, i)

        for i in range(NBUF):
            ws(i)

        plsc.subcore_barrier()

        pltpu.sync_copy(
            acc_sh.at[pl.ds(s * ROWS_PER_TILE, ROWS_PER_TILE)],
            out_hbm.at[c, pl.ds(s * ROWS_PER_TILE, ROWS_PER_TILE)],
        )

    return k


_sc_scatter64 = _make_sc_scatter(64)
_sc_scatter32 = _make_sc_scatter(32)


_BN = 2048


def _tc_matmul(x_p, W1):
    def body(x_ref, w_ref, o_ref):
        o_ref[...] = jnp.dot(x_ref[...], w_ref[...],
                             preferred_element_type=jnp.float32)

    return pl.pallas_call(
        body,
        grid=(NP // _BN,),
        in_specs=[
            pl.BlockSpec((_BN, 128), lambda i: (i, 0)),
            pl.BlockSpec((128, 64), lambda i: (0, 0)),
        ],
        out_specs=pl.BlockSpec((_BN, 64), lambda i: (i, 0)),
        out_shape=jax.ShapeDtypeStruct((NP, 64), jnp.float32),
    )(x_p, W1)


def _tc_scale0(deg_parts, xw):
    def body(deg_ref, xw_ref, g_ref, dinv_ref):
        degT = jnp.transpose(deg_ref[...])
        deg = jnp.sum(degT, axis=1, keepdims=True) + 1.0
        dinv = lax.rsqrt(deg)
        g_ref[...] = xw_ref[...] * dinv
        dinv_ref[...] = dinv

    return pl.pallas_call(
        body,
        grid=(NP // _BN,),
        in_specs=[
            pl.BlockSpec((N_TILES, _BN), lambda i: (0, i)),
            pl.BlockSpec((_BN, 64), lambda i: (i, 0)),
        ],
        out_specs=[
            pl.BlockSpec((_BN, 64), lambda i: (i, 0)),
            pl.BlockSpec((_BN, 1), lambda i: (i, 0)),
        ],
        out_shape=[
            jax.ShapeDtypeStruct((NP, 64), jnp.float32),
            jax.ShapeDtypeStruct((NP, 1), jnp.float32),
        ],
    )(deg_parts, xw)


def _tc_layer(parts, g, dinv, b, Wn, F, Fn):
    def body(p_ref, g_ref, dinv_ref, b_ref, w_ref, o_ref):
        p = p_ref[...]
        t = (p[0] + p[1] + g_ref[...]) * dinv_ref[...]
        h = jnp.maximum(t + b_ref[...], 0.0)
        o_ref[...] = jnp.dot(h * dinv_ref[...], w_ref[...],
                             preferred_element_type=jnp.float32)

    return pl.pallas_call(
        body,
        grid=(NP // _BN,),
        in_specs=[
            pl.BlockSpec((2, _BN, F), lambda i: (0, i, 0)),
            pl.BlockSpec((_BN, F), lambda i: (i, 0)),
            pl.BlockSpec((_BN, 1), lambda i: (i, 0)),
            pl.BlockSpec((1, F), lambda i: (0, 0)),
            pl.BlockSpec((F, Fn), lambda i: (0, 0)),
        ],
        out_specs=pl.BlockSpec((_BN, Fn), lambda i: (i, 0)),
        out_shape=jax.ShapeDtypeStruct((NP, Fn), jnp.float32),
    )(parts, g, dinv, b.reshape(1, F), Wn)


def _tc_final(parts, g, dinv, b, Wcat, bcat):
    def body(p_ref, g_ref, dinv_ref, b_ref, w_ref, bc_ref, o_ref):
        p = p_ref[...]
        t = (p[0] + p[1] + g_ref[...]) * dinv_ref[...]
        h = jnp.maximum(t + b_ref[...], 0.0)
        o_ref[...] = jnp.dot(h, w_ref[...],
                             preferred_element_type=jnp.float32) + bc_ref[...]

    return pl.pallas_call(
        body,
        grid=(NP // _BN,),
        in_specs=[
            pl.BlockSpec((2, _BN, 32), lambda i: (0, i, 0)),
            pl.BlockSpec((_BN, 32), lambda i: (i, 0)),
            pl.BlockSpec((_BN, 1), lambda i: (i, 0)),
            pl.BlockSpec((1, 32), lambda i: (0, 0)),
            pl.BlockSpec((32, 9), lambda i: (0, 0)),
            pl.BlockSpec((1, 9), lambda i: (0, 0)),
        ],
        out_specs=pl.BlockSpec((_BN, 9), lambda i: (i, 0)),
        out_shape=jax.ShapeDtypeStruct((NP, 9), jnp.float32),
    )(parts, g, dinv, b.reshape(1, 32), Wcat, bcat.reshape(1, 9))


def kernel(x, edge_index, W1, b1, W2, b2, W3, b3, We, be, Wh, bh, Wg, bg):
    srcp = edge_index[0].reshape(RW, W_EDGE)
    dstp = edge_index[1].reshape(RW, W_EDGE)
    x_p = jnp.pad(x, ((0, NP - N), (0, 0)))

    xw = _tc_matmul(x_p, W1)
    deg_parts = _sc_degree(dstp, _PAD_IDX).reshape(N_TILES, NP)
    g1, dinv = _tc_scale0(deg_parts, xw)
    p1 = _sc_scatter64(g1, srcp, dstp, _PAD_IDX)
    g2 = _tc_layer(p1, g1, dinv, b1, W2, 64, 64)
    p2 = _sc_scatter64(g2, srcp, dstp, _PAD_IDX)
    g3 = _tc_layer(p2, g2, dinv, b2, W3, 64, 32)
    p3 = _sc_scatter32(g3, srcp, dstp, _PAD_IDX)

    Wcat = jnp.concatenate([We, Wh, Wg], axis=1)
    bcat = jnp.concatenate([be, bh, bg])
    out = _tc_final(p3, g3, dinv, b3, Wcat, bcat)
    return out[:N]


# speedup vs baseline: 43.4740x; 1.0631x over previous
import functools

import numpy as np

import jax
import jax.numpy as jnp
from jax import lax
from jax.experimental import pallas as pl
from jax.experimental.pallas import tpu as pltpu
from jax.experimental.pallas import tpu_sc as plsc

N = 10000
NP = 10240
E = 320000
W_EDGE = 128
RW = E // W_EDGE
PW = 80
PADW = 32 * PW - RW
N_TILES = 32
ROWS_PER_TILE = NP // 16

_PAD_IDX = np.broadcast_to(
    N + (np.arange(PADW * W_EDGE, dtype=np.int32) % (NP - N)).reshape(PADW, 1, W_EDGE),
    (PADW, 2, W_EDGE),
).copy()

_MESH = plsc.VectorSubcoreMesh(core_axis_name="c", subcore_axis_name="s")

_CP = pltpu.CompilerParams(needs_layout_passes=False)
_CP_UNTILED = pltpu.CompilerParams(use_tc_tiling_on_sc=False)


def _load_windows(ei_hbm, pad_hbm, dest, wid):
    @pl.when(wid < N_TILES - 1)
    def _():
        pltpu.sync_copy(ei_hbm.at[pl.ds(wid * PW, PW)], dest)

    @pl.when(wid == N_TILES - 1)
    def _():
        pltpu.sync_copy(ei_hbm.at[pl.ds(RW - (PW - PADW), PW - PADW)],
                        dest.at[pl.ds(0, PW - PADW)])
        pltpu.sync_copy(pad_hbm, dest.at[pl.ds(PW - PADW, PADW)])


@functools.partial(
    pl.kernel,
    out_type=jax.ShapeDtypeStruct((N_TILES, NP // 128, 128), jnp.float32),
    mesh=_MESH,
    compiler_params=_CP,
    scratch_types=[
        pltpu.VMEM((PW, 2, W_EDGE), jnp.int32),
        pltpu.VMEM((NP // 128, 128), jnp.float32),
    ],
)
def _sc_degree(ei_hbm, pad_hbm, out_hbm, idx_v, hist_v):
    c = lax.axis_index("c")
    s = lax.axis_index("s")
    wid = c * 16 + s
    zero16 = jnp.zeros((16,), jnp.float32)
    one16 = jnp.ones((16,), jnp.float32)

    @pl.loop(0, NP // 128)
    def _(r):
        @pl.loop(0, 128, step=16)
        def _(j):
            hist_v[r, pl.ds(j, 16)] = zero16

    _load_windows(ei_hbm, pad_hbm, idx_v, wid)

    @pl.loop(0, PW)
    def _(w):
        @pl.loop(0, W_EDGE, step=16)
        def _(j):
            v = idx_v[w, 1, pl.ds(j, 16)]
            plsc.addupdate_scatter(hist_v, [v >> 7, v & 127], one16)

    pltpu.sync_copy(hist_v, out_hbm.at[wid])


def _make_sc_scatter(F):
    NBUF = 4

    @functools.partial(
        pl.kernel,
        out_type=jax.ShapeDtypeStruct((2, NP, F), jnp.float32),
        mesh=_MESH,
        compiler_params=_CP_UNTILED,
        scratch_types=[
            pltpu.VMEM((PW, 2, W_EDGE), jnp.int32),
            pltpu.VMEM((NBUF, W_EDGE, F), jnp.float32),
            pltpu.VMEM_SHARED((NP, F), jnp.float32),
        ]
        + [pltpu.SemaphoreType.DMA] * (2 * NBUF),
    )
    def k(g_hbm, ei_hbm, pad_hbm, out_hbm, ev, rows,
          acc_sh, *sems):
        gsem = sems[:NBUF]
        ssem = sems[NBUF:]
        c = lax.axis_index("c")
        s = lax.axis_index("s")
        wid = c * 16 + s
        zero16 = jnp.zeros((16,), jnp.float32)

        @pl.loop(0, W_EDGE)
        def _(r):
            @pl.loop(0, F, step=16)
            def _(j):
                rows[0, r, pl.ds(j, 16)] = zero16

        @pl.loop(0, ROWS_PER_TILE, step=W_EDGE)
        def _(kk):
            pltpu.sync_copy(rows.at[0], acc_sh.at[pl.ds(s * ROWS_PER_TILE + kk, W_EDGE)])

        plsc.subcore_barrier()

        _load_windows(ei_hbm, pad_hbm, ev, wid)

        def sg(w, i):
            pltpu.async_copy(g_hbm.at[ev.at[w, 0]], rows.at[i], gsem[i])

        def wg(i):
            pltpu.make_async_copy(g_hbm.at[ev.at[0, 0]], rows.at[i], gsem[i]).wait()

        def ss(w, i):
            pltpu.async_copy(rows.at[i], acc_sh.at[ev.at[w, 1]], ssem[i], add=True)

        def ws(i):
            pltpu.make_async_copy(rows.at[i], acc_sh.at[ev.at[0, 1]], ssem[i]).wait()

        for i in range(NBUF):
            sg(i, i)
        for i in range(NBUF):
            wg(i)
            ss(i, i)

        @pl.loop(NBUF, PW, step=NBUF)
        def _(w):
            for i in range(NBUF):
                ws(i)
                sg(w + i, i)
            for i in range(NBUF):
                wg(i)
                ss(w + i, i)

        for i in range(NBUF):
            ws(i)

        plsc.subcore_barrier()

        pltpu.sync_copy(
            acc_sh.at[pl.ds(s * ROWS_PER_TILE, ROWS_PER_TILE)],
            out_hbm.at[c, pl.ds(s * ROWS_PER_TILE, ROWS_PER_TILE)],
        )

    return k


_sc_scatter64 = _make_sc_scatter(64)
_sc_scatter32 = _make_sc_scatter(32)


_BN = 2048


def _tc_matmul(x_p, W1):
    def body(x_ref, w_ref, o_ref):
        o_ref[...] = jnp.dot(x_ref[...], w_ref[...],
                             preferred_element_type=jnp.float32)

    return pl.pallas_call(
        body,
        grid=(NP // _BN,),
        in_specs=[
            pl.BlockSpec((_BN, 128), lambda i: (i, 0)),
            pl.BlockSpec((128, 64), lambda i: (0, 0)),
        ],
        out_specs=pl.BlockSpec((_BN, 64), lambda i: (i, 0)),
        out_shape=jax.ShapeDtypeStruct((NP, 64), jnp.float32),
    )(x_p, W1)


def _tc_scale0(deg_parts, xw):
    def body(deg_ref, xw_ref, g_ref, dinv_ref):
        degT = jnp.transpose(deg_ref[...])
        deg = jnp.sum(degT, axis=1, keepdims=True) + 1.0
        dinv = lax.rsqrt(deg)
        g_ref[...] = xw_ref[...] * dinv
        dinv_ref[...] = dinv

    return pl.pallas_call(
        body,
        grid=(NP // _BN,),
        in_specs=[
            pl.BlockSpec((N_TILES, _BN), lambda i: (0, i)),
            pl.BlockSpec((_BN, 64), lambda i: (i, 0)),
        ],
        out_specs=[
            pl.BlockSpec((_BN, 64), lambda i: (i, 0)),
            pl.BlockSpec((_BN, 1), lambda i: (i, 0)),
        ],
        out_shape=[
            jax.ShapeDtypeStruct((NP, 64), jnp.float32),
            jax.ShapeDtypeStruct((NP, 1), jnp.float32),
        ],
    )(deg_parts, xw)


def _tc_layer(parts, g, dinv, b, Wn, F, Fn):
    def body(p_ref, g_ref, dinv_ref, b_ref, w_ref, o_ref):
        p = p_ref[...]
        t = (p[0] + p[1] + g_ref[...]) * dinv_ref[...]
        h = jnp.maximum(t + b_ref[...], 0.0)
        o_ref[...] = jnp.dot(h * dinv_ref[...], w_ref[...],
                             preferred_element_type=jnp.float32)

    return pl.pallas_call(
        body,
        grid=(NP // _BN,),
        in_specs=[
            pl.BlockSpec((2, _BN, F), lambda i: (0, i, 0)),
            pl.BlockSpec((_BN, F), lambda i: (i, 0)),
            pl.BlockSpec((_BN, 1), lambda i: (i, 0)),
            pl.BlockSpec((1, F), lambda i: (0, 0)),
            pl.BlockSpec((F, Fn), lambda i: (0, 0)),
        ],
        out_specs=pl.BlockSpec((_BN, Fn), lambda i: (i, 0)),
        out_shape=jax.ShapeDtypeStruct((NP, Fn), jnp.float32),
    )(parts, g, dinv, b.reshape(1, F), Wn)


def _tc_final(parts, g, dinv, b, Wcat, bcat):
    def body(p_ref, g_ref, dinv_ref, b_ref, w_ref, bc_ref, o_ref):
        p = p_ref[...]
        t = (p[0] + p[1] + g_ref[...]) * dinv_ref[...]
        h = jnp.maximum(t + b_ref[...], 0.0)
        o_ref[...] = jnp.dot(h, w_ref[...],
                             preferred_element_type=jnp.float32) + bc_ref[...]

    return pl.pallas_call(
        body,
        grid=(NP // _BN,),
        in_specs=[
            pl.BlockSpec((2, _BN, 32), lambda i: (0, i, 0)),
            pl.BlockSpec((_BN, 32), lambda i: (i, 0)),
            pl.BlockSpec((_BN, 1), lambda i: (i, 0)),
            pl.BlockSpec((1, 32), lambda i: (0, 0)),
            pl.BlockSpec((32, 9), lambda i: (0, 0)),
            pl.BlockSpec((1, 9), lambda i: (0, 0)),
        ],
        out_specs=pl.BlockSpec((_BN, 9), lambda i: (i, 0)),
        out_shape=jax.ShapeDtypeStruct((NP, 9), jnp.float32),
    )(parts, g, dinv, b.reshape(1, 32), Wcat, bcat.reshape(1, 9))


def kernel(x, edge_index, W1, b1, W2, b2, W3, b3, We, be, Wh, bh, Wg, bg):
    ei3 = jnp.transpose(edge_index.reshape(2, RW, W_EDGE), (1, 0, 2))
    x_p = jnp.pad(x, ((0, NP - N), (0, 0)))

    xw = _tc_matmul(x_p, W1)
    deg_parts = _sc_degree(ei3, _PAD_IDX).reshape(N_TILES, NP)
    g1, dinv = _tc_scale0(deg_parts, xw)
    p1 = _sc_scatter64(g1, ei3, _PAD_IDX)
    g2 = _tc_layer(p1, g1, dinv, b1, W2, 64, 64)
    p2 = _sc_scatter64(g2, ei3, _PAD_IDX)
    g3 = _tc_layer(p2, g2, dinv, b2, W3, 64, 32)
    p3 = _sc_scatter32(g3, ei3, _PAD_IDX)

    Wcat = jnp.concatenate([We, Wh, Wg], axis=1)
    bcat = jnp.concatenate([be, bh, bg])
    out = _tc_final(p3, g3, dinv, b3, Wcat, bcat)
    return out[:N]


# speedup vs baseline: 51.2928x; 1.1799x over previous
import functools

import numpy as np

import jax
import jax.numpy as jnp
from jax import lax
from jax.experimental import pallas as pl
from jax.experimental.pallas import tpu as pltpu
from jax.experimental.pallas import tpu_sc as plsc

N = 10000
NP = 10240
E = 320000
W_EDGE = 128
RW = E // W_EDGE
PW = 80
PADW = 32 * PW - RW
N_TILES = 32
ROWS_PER_TILE = NP // 16

_PAD_IDX = np.broadcast_to(
    N + (np.arange(PADW * W_EDGE, dtype=np.int32) % (NP - N)).reshape(PADW, 1, W_EDGE),
    (PADW, 2, W_EDGE),
).copy()

_MESH = plsc.VectorSubcoreMesh(core_axis_name="c", subcore_axis_name="s")

_CP = pltpu.CompilerParams(needs_layout_passes=False)
_CP_UNTILED = pltpu.CompilerParams(use_tc_tiling_on_sc=False)


def _load_windows(ei_hbm, pad_hbm, dest, wid):
    @pl.when(wid < N_TILES - 1)
    def _():
        pltpu.sync_copy(ei_hbm.at[pl.ds(wid * PW, PW)], dest)

    @pl.when(wid == N_TILES - 1)
    def _():
        pltpu.sync_copy(ei_hbm.at[pl.ds(RW - (PW - PADW), PW - PADW)],
                        dest.at[pl.ds(0, PW - PADW)])
        pltpu.sync_copy(pad_hbm, dest.at[pl.ds(PW - PADW, PADW)])


NH = NP // 2
NQ = NP // 4
HR = NP // 128
HR_T = HR // 16
P2_T = NH // 16
P4_T = NQ // 16


@functools.partial(
    pl.kernel,
    out_type=[
        jax.ShapeDtypeStruct((2, NH, 128), jnp.float32),
        jax.ShapeDtypeStruct((2, NQ, 128), jnp.float32),
    ],
    mesh=_MESH,
    compiler_params=_CP,
    scratch_types=[
        pltpu.VMEM((PW, 2, W_EDGE), jnp.int32),
        pltpu.VMEM((HR, 128), jnp.float32),
        pltpu.VMEM((HR_T, 128), jnp.float32),
        pltpu.VMEM((HR_T, 128), jnp.float32),
        pltpu.VMEM((P2_T, 128), jnp.float32),
        pltpu.VMEM((P4_T, 128), jnp.float32),
        pltpu.VMEM_SHARED((16, HR, 128), jnp.float32),
    ],
)
def _sc_degree(ei_hbm, pad_hbm, p2_hbm, p4_hbm, idx_v, hist_v, hsum, tmp,
               e2, e4, stage):
    c = lax.axis_index("c")
    s = lax.axis_index("s")
    wid = c * 16 + s
    zero16 = jnp.zeros((16,), jnp.float32)
    one16 = jnp.ones((16,), jnp.float32)

    @pl.loop(0, HR)
    def _(r):
        @pl.loop(0, 128, step=16)
        def _(j):
            hist_v[r, pl.ds(j, 16)] = zero16

    _load_windows(ei_hbm, pad_hbm, idx_v, wid)

    @pl.loop(0, PW)
    def _(w):
        @pl.loop(0, W_EDGE, step=16)
        def _(j):
            v = idx_v[w, 1, pl.ds(j, 16)]
            plsc.addupdate_scatter(hist_v, [v >> 7, v & 127], one16)

    pltpu.sync_copy(hist_v, stage.at[s])
    plsc.subcore_barrier()

    for r in range(HR_T):
        for k in range(8):
            hsum[r, pl.ds(k * 16, 16)] = zero16

    @pl.loop(0, 16)
    def _(j):
        pltpu.sync_copy(stage.at[j, pl.ds(s * HR_T, HR_T)], tmp)
        for r in range(HR_T):
            for k in range(8):
                sl = (r, pl.ds(k * 16, 16))
                hsum[sl] = hsum[sl] + tmp[sl]

    def bcast(val):
        return jnp.full((16,), val, jnp.int32)

    def deg_of(n):
        return plsc.load_gather(hsum, [bcast(n >> 7), bcast(n & 127)])

    @pl.loop(0, P2_T)
    def _(r):
        va = deg_of(2 * r)
        vb = deg_of(2 * r + 1)
        for k in range(4):
            e2[r, pl.ds(k * 16, 16)] = va
        for k in range(4, 8):
            e2[r, pl.ds(k * 16, 16)] = vb

    @pl.loop(0, P4_T)
    def _(q):
        for i in range(4):
            vv = deg_of(4 * q + i)
            for k in range(2):
                e4[q, pl.ds((2 * i + k) * 16, 16)] = vv

    pltpu.sync_copy(e2, p2_hbm.at[c, pl.ds(s * P2_T, P2_T)])
    pltpu.sync_copy(e4, p4_hbm.at[c, pl.ds(s * P4_T, P4_T)])


def _make_sc_scatter(F):
    NBUF = 4

    @functools.partial(
        pl.kernel,
        out_type=jax.ShapeDtypeStruct((2, NP, F), jnp.float32),
        mesh=_MESH,
        compiler_params=_CP_UNTILED,
        scratch_types=[
            pltpu.VMEM((PW, 2, W_EDGE), jnp.int32),
            pltpu.VMEM((NBUF, W_EDGE, F), jnp.float32),
            pltpu.VMEM_SHARED((NP, F), jnp.float32),
        ]
        + [pltpu.SemaphoreType.DMA] * (2 * NBUF),
    )
    def k(g_hbm, ei_hbm, pad_hbm, out_hbm, ev, rows,
          acc_sh, *sems):
        gsem = sems[:NBUF]
        ssem = sems[NBUF:]
        c = lax.axis_index("c")
        s = lax.axis_index("s")
        wid = c * 16 + s
        zero16 = jnp.zeros((16,), jnp.float32)

        @pl.loop(0, W_EDGE)
        def _(r):
            @pl.loop(0, F, step=16)
            def _(j):
                rows[0, r, pl.ds(j, 16)] = zero16

        @pl.loop(0, ROWS_PER_TILE, step=W_EDGE)
        def _(kk):
            pltpu.sync_copy(rows.at[0], acc_sh.at[pl.ds(s * ROWS_PER_TILE + kk, W_EDGE)])

        plsc.subcore_barrier()

        _load_windows(ei_hbm, pad_hbm, ev, wid)

        def sg(w, i):
            pltpu.async_copy(g_hbm.at[ev.at[w, 0]], rows.at[i], gsem[i])

        def wg(i):
            pltpu.make_async_copy(g_hbm.at[ev.at[0, 0]], rows.at[i], gsem[i]).wait()

        def ss(w, i):
            pltpu.async_copy(rows.at[i], acc_sh.at[ev.at[w, 1]], ssem[i], add=True)

        def ws(i):
            pltpu.make_async_copy(rows.at[i], acc_sh.at[ev.at[0, 1]], ssem[i]).wait()

        for i in range(NBUF):
            sg(i, i)
        for i in range(NBUF):
            wg(i)
            ss(i, i)

        @pl.loop(NBUF, PW, step=NBUF)
        def _(w):
            for i in range(NBUF):
                ws(i)
                sg(w + i, i)
            for i in range(NBUF):
                wg(i)
                ss(w + i, i)

        for i in range(NBUF):
            ws(i)

        plsc.subcore_barrier()

        pltpu.sync_copy(
            acc_sh.at[pl.ds(s * ROWS_PER_TILE, ROWS_PER_TILE)],
            out_hbm.at[c, pl.ds(s * ROWS_PER_TILE, ROWS_PER_TILE)],
        )

    return k


_sc_scatter64 = _make_sc_scatter(64)
_sc_scatter32 = _make_sc_scatter(32)


_BH = 1024
_BQ = 512


def _tc_matmul(xP, W1s):
    def body(x_ref, w_ref, o_ref):
        o_ref[...] = jnp.dot(x_ref[...], w_ref[...],
                             preferred_element_type=jnp.float32)

    return pl.pallas_call(
        body,
        grid=(NH // _BH,),
        in_specs=[
            pl.BlockSpec((_BH, 256), lambda i: (i, 0)),
            pl.BlockSpec((256, 128), lambda i: (0, 0)),
        ],
        out_specs=pl.BlockSpec((_BH, 128), lambda i: (i, 0)),
        out_shape=jax.ShapeDtypeStruct((NH, 128), jnp.float32),
    )(xP, W1s)


def _tc_scale0(degP2, degP4, xwP):
    def body(p2_ref, p4_ref, xw_ref, g_ref, d2_ref, d4_ref):
        a = p2_ref[...]
        d2 = lax.rsqrt(a[0] + a[1] + 1.0)
        b = p4_ref[...]
        d4 = lax.rsqrt(b[0] + b[1] + 1.0)
        d2_ref[...] = d2
        d4_ref[...] = d4
        g_ref[...] = xw_ref[...] * d2

    return pl.pallas_call(
        body,
        grid=(NH // _BH,),
        in_specs=[
            pl.BlockSpec((2, _BH, 128), lambda i: (0, i, 0)),
            pl.BlockSpec((2, _BQ, 128), lambda i: (0, i, 0)),
            pl.BlockSpec((_BH, 128), lambda i: (i, 0)),
        ],
        out_specs=[
            pl.BlockSpec((_BH, 128), lambda i: (i, 0)),
            pl.BlockSpec((_BH, 128), lambda i: (i, 0)),
            pl.BlockSpec((_BQ, 128), lambda i: (i, 0)),
        ],
        out_shape=[
            jax.ShapeDtypeStruct((NH, 128), jnp.float32),
            jax.ShapeDtypeStruct((NH, 128), jnp.float32),
            jax.ShapeDtypeStruct((NQ, 128), jnp.float32),
        ],
    )(degP2, degP4, xwP)


def _tc_layer(partsP, gP, dinvP2, bP, Ws, Kout):
    def body(p_ref, g_ref, d_ref, b_ref, w_ref, o_ref):
        p = p_ref[...]
        d = d_ref[...]
        h = jnp.maximum((p[0] + p[1] + g_ref[...]) * d + b_ref[...], 0.0)
        o_ref[...] = jnp.dot(h * d, w_ref[...],
                             preferred_element_type=jnp.float32)

    return pl.pallas_call(
        body,
        grid=(NH // _BH,),
        in_specs=[
            pl.BlockSpec((2, _BH, 128), lambda i: (0, i, 0)),
            pl.BlockSpec((_BH, 128), lambda i: (i, 0)),
            pl.BlockSpec((_BH, 128), lambda i: (i, 0)),
            pl.BlockSpec((1, 128), lambda i: (0, 0)),
            pl.BlockSpec((128, Kout), lambda i: (0, 0)),
        ],
        out_specs=pl.BlockSpec((_BH, Kout), lambda i: (i, 0)),
        out_shape=jax.ShapeDtypeStruct((NH, Kout), jnp.float32),
    )(partsP, gP, dinvP2, bP.reshape(1, 128), Ws)


def _tc_final(partsQ, gQ, dinvP4, bQ, WcatQ, bcatQ):
    def body(p_ref, g_ref, d_ref, b_ref, w_ref, bc_ref, o_ref):
        p = p_ref[...]
        h = jnp.maximum((p[0] + p[1] + g_ref[...]) * d_ref[...] + b_ref[...],
                        0.0)
        o_ref[...] = jnp.dot(h, w_ref[...],
                             preferred_element_type=jnp.float32) + bc_ref[...]

    return pl.pallas_call(
        body,
        grid=(NQ // _BQ,),
        in_specs=[
            pl.BlockSpec((2, _BQ, 128), lambda i: (0, i, 0)),
            pl.BlockSpec((_BQ, 128), lambda i: (i, 0)),
            pl.BlockSpec((_BQ, 128), lambda i: (i, 0)),
            pl.BlockSpec((1, 128), lambda i: (0, 0)),
            pl.BlockSpec((128, 36), lambda i: (0, 0)),
            pl.BlockSpec((1, 36), lambda i: (0, 0)),
        ],
        out_specs=pl.BlockSpec((_BQ, 36), lambda i: (i, 0)),
        out_shape=jax.ShapeDtypeStruct((NQ, 36), jnp.float32),
    )(partsQ, gQ, dinvP4, bQ.reshape(1, 128), WcatQ, bcatQ.reshape(1, 36))


def _blockdiag(W, n):
    K, M = W.shape
    out = jnp.zeros((n * K, n * M), W.dtype)
    for i in range(n):
        out = out.at[i * K:(i + 1) * K, i * M:(i + 1) * M].set(W)
    return out


def kernel(x, edge_index, W1, b1, W2, b2, W3, b3, We, be, Wh, bh, Wg, bg):
    ei3 = jnp.transpose(edge_index.reshape(2, RW, W_EDGE), (1, 0, 2))
    xP = jnp.pad(x, ((0, NP - N), (0, 0))).reshape(NH, 256)

    W1s = _blockdiag(W1, 2)
    W2s = _blockdiag(W2, 2)
    W3s = _blockdiag(W3, 2)
    b1P = jnp.concatenate([b1, b1])
    b2P = jnp.concatenate([b2, b2])

    xwP = _tc_matmul(xP, W1s)
    degP2, degP4 = _sc_degree(ei3, _PAD_IDX)
    g1P, d2, d4 = _tc_scale0(degP2, degP4, xwP)

    p1 = _sc_scatter64(g1P.reshape(NP, 64), ei3, _PAD_IDX)
    g2P = _tc_layer(p1.reshape(2, NH, 128), g1P, d2, b1P, W2s, 128)
    p2 = _sc_scatter64(g2P.reshape(NP, 64), ei3, _PAD_IDX)
    g3 = _tc_layer(p2.reshape(2, NH, 128), g2P, d2, b2P, W3s, 64)

    g3d = g3.reshape(NP, 32)
    p3 = _sc_scatter32(g3d, ei3, _PAD_IDX)

    Wcat = jnp.concatenate([We, Wh, Wg], axis=1)
    WcatQ = _blockdiag(Wcat, 4)
    bcat = jnp.concatenate([be, bh, bg])
    b3Q = jnp.concatenate([b3] * 4)
    bcatQ = jnp.concatenate([bcat] * 4)

    out4 = _tc_final(p3.reshape(2, NQ, 128), g3d.reshape(NQ, 128),
                     d4, b3Q, WcatQ, bcatQ)
    return out4.reshape(NP, 9)[:N]


# speedup vs baseline: 53.9661x; 1.0521x over previous
import functools

import numpy as np

import jax
import jax.numpy as jnp
from jax import lax
from jax.experimental import pallas as pl
from jax.experimental.pallas import tpu as pltpu
from jax.experimental.pallas import tpu_sc as plsc

N = 10000
NP = 10240
E = 320000
W_EDGE = 128
RW = E // W_EDGE
PW = 80
PADW = 32 * PW - RW
N_TILES = 32
ROWS_PER_TILE = NP // 16

_PAD_IDX = np.broadcast_to(
    N + (np.arange(PADW * W_EDGE, dtype=np.int32) % (NP - N)).reshape(PADW, 1, W_EDGE),
    (PADW, 2, W_EDGE),
).copy()

_MESH = plsc.VectorSubcoreMesh(core_axis_name="c", subcore_axis_name="s")

_CP = pltpu.CompilerParams(needs_layout_passes=False)
_CP_UNTILED = pltpu.CompilerParams(use_tc_tiling_on_sc=False)


def _load_windows(ei_hbm, pad_hbm, dest, wid):
    @pl.when(wid < N_TILES - 1)
    def _():
        pltpu.sync_copy(ei_hbm.at[pl.ds(wid * PW, PW)], dest)

    @pl.when(wid == N_TILES - 1)
    def _():
        pltpu.sync_copy(ei_hbm.at[pl.ds(RW - (PW - PADW), PW - PADW)],
                        dest.at[pl.ds(0, PW - PADW)])
        pltpu.sync_copy(pad_hbm, dest.at[pl.ds(PW - PADW, PADW)])


NH = NP // 2
NQ = NP // 4
HR = NP // 128
HR_T = HR // 16
P2_T = NH // 16
P4_T = NQ // 16


@functools.partial(
    pl.kernel,
    out_type=[
        jax.ShapeDtypeStruct((2, NH, 128), jnp.float32),
        jax.ShapeDtypeStruct((2, NQ, 128), jnp.float32),
    ],
    mesh=_MESH,
    compiler_params=_CP,
    scratch_types=[
        pltpu.VMEM((PW, 2, W_EDGE), jnp.int32),
        pltpu.VMEM((HR, 128), jnp.float32),
        pltpu.VMEM((HR_T, 128), jnp.float32),
        pltpu.VMEM((HR_T, 128), jnp.float32),
        pltpu.VMEM((P2_T, 128), jnp.float32),
        pltpu.VMEM((P4_T, 128), jnp.float32),
        pltpu.VMEM_SHARED((16, HR, 128), jnp.float32),
    ],
)
def _sc_degree(ei_hbm, pad_hbm, p2_hbm, p4_hbm, idx_v, hist_v, hsum, tmp,
               e2, e4, stage):
    c = lax.axis_index("c")
    s = lax.axis_index("s")
    wid = c * 16 + s
    zero16 = jnp.zeros((16,), jnp.float32)
    one16 = jnp.ones((16,), jnp.float32)

    @pl.loop(0, HR)
    def _(r):
        @pl.loop(0, 128, step=16)
        def _(j):
            hist_v[r, pl.ds(j, 16)] = zero16

    _load_windows(ei_hbm, pad_hbm, idx_v, wid)

    @pl.loop(0, PW)
    def _(w):
        @pl.loop(0, W_EDGE, step=16)
        def _(j):
            v = idx_v[w, 1, pl.ds(j, 16)]
            plsc.addupdate_scatter(hist_v, [v >> 7, v & 127], one16)

    pltpu.sync_copy(hist_v, stage.at[s])
    plsc.subcore_barrier()

    for r in range(HR_T):
        for k in range(8):
            hsum[r, pl.ds(k * 16, 16)] = zero16

    @pl.loop(0, 16)
    def _(j):
        pltpu.sync_copy(stage.at[j, pl.ds(s * HR_T, HR_T)], tmp)
        for r in range(HR_T):
            for k in range(8):
                sl = (r, pl.ds(k * 16, 16))
                hsum[sl] = hsum[sl] + tmp[sl]

    def bcast(val):
        return jnp.full((16,), val, jnp.int32)

    def deg_of(n):
        return plsc.load_gather(hsum, [bcast(n >> 7), bcast(n & 127)])

    @pl.loop(0, P2_T)
    def _(r):
        va = deg_of(2 * r)
        vb = deg_of(2 * r + 1)
        for k in range(4):
            e2[r, pl.ds(k * 16, 16)] = va
        for k in range(4, 8):
            e2[r, pl.ds(k * 16, 16)] = vb

    @pl.loop(0, P4_T)
    def _(q):
        for i in range(4):
            vv = deg_of(4 * q + i)
            for k in range(2):
                e4[q, pl.ds((2 * i + k) * 16, 16)] = vv

    pltpu.sync_copy(e2, p2_hbm.at[c, pl.ds(s * P2_T, P2_T)])
    pltpu.sync_copy(e4, p4_hbm.at[c, pl.ds(s * P4_T, P4_T)])


def _make_sc_scatter(F):
    NBUF = 8

    @functools.partial(
        pl.kernel,
        out_type=jax.ShapeDtypeStruct((2, NP, F), jnp.float32),
        mesh=_MESH,
        compiler_params=_CP_UNTILED,
        scratch_types=[
            pltpu.VMEM((PW, 2, W_EDGE), jnp.int32),
            pltpu.VMEM((NBUF, W_EDGE, F), jnp.float32),
            pltpu.VMEM_SHARED((NP, F), jnp.float32),
        ]
        + [pltpu.SemaphoreType.DMA] * (2 * NBUF),
    )
    def k(g_hbm, ei_hbm, pad_hbm, out_hbm, ev, rows,
          acc_sh, *sems):
        gsem = sems[:NBUF]
        ssem = sems[NBUF:]
        c = lax.axis_index("c")
        s = lax.axis_index("s")
        wid = c * 16 + s
        zero16 = jnp.zeros((16,), jnp.float32)

        @pl.loop(0, W_EDGE)
        def _(r):
            @pl.loop(0, F, step=16)
            def _(j):
                rows[0, r, pl.ds(j, 16)] = zero16

        @pl.loop(0, ROWS_PER_TILE, step=W_EDGE)
        def _(kk):
            pltpu.sync_copy(rows.at[0], acc_sh.at[pl.ds(s * ROWS_PER_TILE + kk, W_EDGE)])

        plsc.subcore_barrier()

        _load_windows(ei_hbm, pad_hbm, ev, wid)

        def sg(w, i):
            pltpu.async_copy(g_hbm.at[ev.at[w, 0]], rows.at[i], gsem[i])

        def wg(i):
            pltpu.make_async_copy(g_hbm.at[ev.at[0, 0]], rows.at[i], gsem[i]).wait()

        def ss(w, i):
            pltpu.async_copy(rows.at[i], acc_sh.at[ev.at[w, 1]], ssem[i], add=True)

        def ws(i):
            pltpu.make_async_copy(rows.at[i], acc_sh.at[ev.at[0, 1]], ssem[i]).wait()

        for i in range(NBUF):
            sg(i, i)
        for i in range(NBUF):
            wg(i)
            ss(i, i)

        @pl.loop(NBUF, PW, step=NBUF)
        def _(w):
            for i in range(NBUF):
                ws(i)
                sg(w + i, i)
            for i in range(NBUF):
                wg(i)
                ss(w + i, i)

        for i in range(NBUF):
            ws(i)

        plsc.subcore_barrier()

        pltpu.sync_copy(
            acc_sh.at[pl.ds(s * ROWS_PER_TILE, ROWS_PER_TILE)],
            out_hbm.at[c, pl.ds(s * ROWS_PER_TILE, ROWS_PER_TILE)],
        )

    return k


_sc_scatter64 = _make_sc_scatter(64)
_sc_scatter32 = _make_sc_scatter(32)


_BH = 1024
_BQ = 512


def _tc_matmul(xP, W1s):
    def body(x_ref, w_ref, o_ref):
        o_ref[...] = jnp.dot(x_ref[...], w_ref[...],
                             preferred_element_type=jnp.float32)

    return pl.pallas_call(
        body,
        grid=(NH // _BH,),
        in_specs=[
            pl.BlockSpec((_BH, 256), lambda i: (i, 0)),
            pl.BlockSpec((256, 128), lambda i: (0, 0)),
        ],
        out_specs=pl.BlockSpec((_BH, 128), lambda i: (i, 0)),
        out_shape=jax.ShapeDtypeStruct((NH, 128), jnp.float32),
    )(xP, W1s)


def _tc_scale0(degP2, degP4, xwP):
    def body(p2_ref, p4_ref, xw_ref, g_ref, d2_ref, d4_ref):
        a = p2_ref[...]
        d2 = lax.rsqrt(a[0] + a[1] + 1.0)
        b = p4_ref[...]
        d4 = lax.rsqrt(b[0] + b[1] + 1.0)
        d2_ref[...] = d2
        d4_ref[...] = d4
        g_ref[...] = xw_ref[...] * d2

    return pl.pallas_call(
        body,
        grid=(NH // _BH,),
        in_specs=[
            pl.BlockSpec((2, _BH, 128), lambda i: (0, i, 0)),
            pl.BlockSpec((2, _BQ, 128), lambda i: (0, i, 0)),
            pl.BlockSpec((_BH, 128), lambda i: (i, 0)),
        ],
        out_specs=[
            pl.BlockSpec((_BH, 128), lambda i: (i, 0)),
            pl.BlockSpec((_BH, 128), lambda i: (i, 0)),
            pl.BlockSpec((_BQ, 128), lambda i: (i, 0)),
        ],
        out_shape=[
            jax.ShapeDtypeStruct((NH, 128), jnp.float32),
            jax.ShapeDtypeStruct((NH, 128), jnp.float32),
            jax.ShapeDtypeStruct((NQ, 128), jnp.float32),
        ],
    )(degP2, degP4, xwP)


def _tc_layer(partsP, gP, dinvP2, bP, Ws, Kout):
    def body(p_ref, g_ref, d_ref, b_ref, w_ref, o_ref):
        p = p_ref[...]
        d = d_ref[...]
        h = jnp.maximum((p[0] + p[1] + g_ref[...]) * d + b_ref[...], 0.0)
        o_ref[...] = jnp.dot(h * d, w_ref[...],
                             preferred_element_type=jnp.float32)

    return pl.pallas_call(
        body,
        grid=(NH // _BH,),
        in_specs=[
            pl.BlockSpec((2, _BH, 128), lambda i: (0, i, 0)),
            pl.BlockSpec((_BH, 128), lambda i: (i, 0)),
            pl.BlockSpec((_BH, 128), lambda i: (i, 0)),
            pl.BlockSpec((1, 128), lambda i: (0, 0)),
            pl.BlockSpec((128, Kout), lambda i: (0, 0)),
        ],
        out_specs=pl.BlockSpec((_BH, Kout), lambda i: (i, 0)),
        out_shape=jax.ShapeDtypeStruct((NH, Kout), jnp.float32),
    )(partsP, gP, dinvP2, bP.reshape(1, 128), Ws)


def _tc_final(partsQ, gQ, dinvP4, bQ, WcatQ, bcatQ):
    def body(p_ref, g_ref, d_ref, b_ref, w_ref, bc_ref, o_ref):
        p = p_ref[...]
        h = jnp.maximum((p[0] + p[1] + g_ref[...]) * d_ref[...] + b_ref[...],
                        0.0)
        o_ref[...] = jnp.dot(h, w_ref[...],
                             preferred_element_type=jnp.float32) + bc_ref[...]

    return pl.pallas_call(
        body,
        grid=(NQ // _BQ,),
        in_specs=[
            pl.BlockSpec((2, _BQ, 128), lambda i: (0, i, 0)),
            pl.BlockSpec((_BQ, 128), lambda i: (i, 0)),
            pl.BlockSpec((_BQ, 128), lambda i: (i, 0)),
            pl.BlockSpec((1, 128), lambda i: (0, 0)),
            pl.BlockSpec((128, 36), lambda i: (0, 0)),
            pl.BlockSpec((1, 36), lambda i: (0, 0)),
        ],
        out_specs=pl.BlockSpec((_BQ, 36), lambda i: (i, 0)),
        out_shape=jax.ShapeDtypeStruct((N // 4, 36), jnp.float32),
    )(partsQ, gQ, dinvP4, bQ.reshape(1, 128), WcatQ, bcatQ.reshape(1, 36))


def _blockdiag(W, n):
    K, M = W.shape
    out = jnp.zeros((n * K, n * M), W.dtype)
    for i in range(n):
        out = out.at[i * K:(i + 1) * K, i * M:(i + 1) * M].set(W)
    return out


def kernel(x, edge_index, W1, b1, W2, b2, W3, b3, We, be, Wh, bh, Wg, bg):
    ei3 = jnp.transpose(edge_index.reshape(2, RW, W_EDGE), (1, 0, 2))
    xP = jnp.pad(x, ((0, NP - N), (0, 0))).reshape(NH, 256)

    W1s = _blockdiag(W1, 2)
    W2s = _blockdiag(W2, 2)
    W3s = _blockdiag(W3, 2)
    b1P = jnp.concatenate([b1, b1])
    b2P = jnp.concatenate([b2, b2])

    xwP = _tc_matmul(xP, W1s)
    degP2, degP4 = _sc_degree(ei3, _PAD_IDX)
    g1P, d2, d4 = _tc_scale0(degP2, degP4, xwP)

    p1 = _sc_scatter64(g1P.reshape(NP, 64), ei3, _PAD_IDX)
    g2P = _tc_layer(p1.reshape(2, NH, 128), g1P, d2, b1P, W2s, 128)
    p2 = _sc_scatter64(g2P.reshape(NP, 64), ei3, _PAD_IDX)
    g3 = _tc_layer(p2.reshape(2, NH, 128), g2P, d2, b2P, W3s, 64)

    g3d = g3.reshape(NP, 32)
    p3 = _sc_scatter32(g3d, ei3, _PAD_IDX)

    Wcat = jnp.concatenate([We, Wh, Wg], axis=1)
    WcatQ = _blockdiag(Wcat, 4)
    bcat = jnp.concatenate([be, bh, bg])
    b3Q = jnp.concatenate([b3] * 4)
    bcatQ = jnp.concatenate([bcat] * 4)

    out4 = _tc_final(p3.reshape(2, NQ, 128), g3d.reshape(NQ, 128),
                     d4, b3Q, WcatQ, bcatQ)
    return out4.reshape(N, 9)


# speedup vs baseline: 55.2156x; 1.0232x over previous
import functools

import numpy as np

import jax
import jax.numpy as jnp
from jax import lax
from jax.experimental import pallas as pl
from jax.experimental.pallas import tpu as pltpu
from jax.experimental.pallas import tpu_sc as plsc

N = 10000
NP = 10240
E = 320000
W_EDGE = 128
RW = E // W_EDGE
PW = 80
PADW = 32 * PW - RW
N_TILES = 32
ROWS_PER_TILE = NP // 16

_PAD_IDX = np.broadcast_to(
    N + (np.arange(PADW * W_EDGE, dtype=np.int32) % (NP - N)).reshape(PADW, 1, W_EDGE),
    (PADW, 2, W_EDGE),
).copy()

_MESH = plsc.VectorSubcoreMesh(core_axis_name="c", subcore_axis_name="s")

_CP = pltpu.CompilerParams(needs_layout_passes=False)
_CP_UNTILED = pltpu.CompilerParams(use_tc_tiling_on_sc=False)


def _load_windows(ei_hbm, pad_hbm, dest, wid, sem=None):
    cp = pltpu.sync_copy if sem is None else (
        lambda a, b: pltpu.async_copy(a, b, sem))

    @pl.when(wid < N_TILES - 1)
    def _():
        cp(ei_hbm.at[pl.ds(wid * PW, PW)], dest)

    @pl.when(wid == N_TILES - 1)
    def _():
        cp(ei_hbm.at[pl.ds(RW - (PW - PADW), PW - PADW)],
           dest.at[pl.ds(0, PW - PADW)])
        cp(pad_hbm, dest.at[pl.ds(PW - PADW, PADW)])


NH = NP // 2
NQ = NP // 4
HR = NP // 128
HR_T = HR // 16
P2_T = NH // 16
P4_T = NQ // 16


@functools.partial(
    pl.kernel,
    out_type=[
        jax.ShapeDtypeStruct((2, NH, 128), jnp.float32),
        jax.ShapeDtypeStruct((2, NQ, 128), jnp.float32),
    ],
    mesh=_MESH,
    compiler_params=_CP,
    scratch_types=[
        pltpu.VMEM((PW, 2, W_EDGE), jnp.int32),
        pltpu.VMEM((HR, 128), jnp.float32),
        pltpu.VMEM((HR_T, 128), jnp.float32),
        pltpu.VMEM((HR_T, 128), jnp.float32),
        pltpu.VMEM((HR_T, 128), jnp.float32),
        pltpu.VMEM((P2_T, 128), jnp.float32),
        pltpu.VMEM((P4_T, 128), jnp.float32),
        pltpu.VMEM_SHARED((16, HR, 128), jnp.float32),
        pltpu.SemaphoreType.DMA,
        pltpu.SemaphoreType.DMA,
        pltpu.SemaphoreType.DMA,
    ],
)
def _sc_degree(ei_hbm, pad_hbm, p2_hbm, p4_hbm, idx_v, hist_v, hsum, tmp0,
               tmp1, e2, e4, stage, lsem, t0sem, t1sem):
    c = lax.axis_index("c")
    s = lax.axis_index("s")
    wid = c * 16 + s
    zero16 = jnp.zeros((16,), jnp.float32)
    one16 = jnp.ones((16,), jnp.float32)
    tsem = [t0sem, t1sem]
    tbuf = [tmp0, tmp1]

    @pl.loop(0, HR)
    def _(r):
        for j in range(0, 128, 16):
            hist_v[r, pl.ds(j, 16)] = zero16

    _load_windows(ei_hbm, pad_hbm, idx_v, wid)

    @pl.loop(0, PW)
    def _(w):
        for j in range(0, W_EDGE, 16):
            v = idx_v[w, 1, pl.ds(j, 16)]
            plsc.addupdate_scatter(hist_v, [v >> 7, v & 127], one16)

    pltpu.sync_copy(hist_v, stage.at[s])
    plsc.subcore_barrier()

    def stage_cp(j, i):
        pltpu.async_copy(stage.at[j, pl.ds(s * HR_T, HR_T)], tbuf[i], tsem[i])

    def stage_wait(i):
        pltpu.make_async_copy(stage.at[0, pl.ds(s * HR_T, HR_T)], tbuf[i],
                              tsem[i]).wait()

    stage_cp(0, 0)
    stage_cp(1, 1)
    for j in range(16):
        i = j % 2
        stage_wait(i)
        for r in range(HR_T):
            for k in range(8):
                sl = (r, pl.ds(k * 16, 16))
                if j == 0:
                    hsum[sl] = tbuf[i][sl]
                else:
                    hsum[sl] = hsum[sl] + tbuf[i][sl]
        if j + 2 < 16:
            stage_cp(j + 2, i)

    def bcast(val):
        return jnp.full((16,), val, jnp.int32)

    def deg_of(n):
        return plsc.load_gather(hsum, [bcast(n >> 7), bcast(n & 127)])

    @pl.loop(0, P2_T)
    def _(r):
        va = deg_of(2 * r)
        vb = deg_of(2 * r + 1)
        for k in range(4):
            e2[r, pl.ds(k * 16, 16)] = va
        for k in range(4, 8):
            e2[r, pl.ds(k * 16, 16)] = vb

    pltpu.async_copy(e2, p2_hbm.at[c, pl.ds(s * P2_T, P2_T)], lsem)

    @pl.loop(0, P4_T)
    def _(q):
        for i in range(4):
            vv = deg_of(4 * q + i)
            for k in range(2):
                e4[q, pl.ds((2 * i + k) * 16, 16)] = vv

    pltpu.make_async_copy(e2, p2_hbm.at[c, pl.ds(s * P2_T, P2_T)], lsem).wait()
    pltpu.sync_copy(e4, p4_hbm.at[c, pl.ds(s * P4_T, P4_T)])


def _make_sc_scatter(F):
    NBUF = 8

    @functools.partial(
        pl.kernel,
        out_type=jax.ShapeDtypeStruct((2, NP, F), jnp.float32),
        mesh=_MESH,
        compiler_params=_CP_UNTILED,
        scratch_types=[
            pltpu.VMEM((PW, 2, W_EDGE), jnp.int32),
            pltpu.VMEM((NBUF, W_EDGE, F), jnp.float32),
            pltpu.VMEM_SHARED((NP, F), jnp.float32),
        ]
        + [pltpu.SemaphoreType.DMA] * (2 * NBUF + 1),
    )
    def k(g_hbm, ei_hbm, pad_hbm, out_hbm, ev, rows,
          acc_sh, *sems):
        gsem = sems[:NBUF]
        ssem = sems[NBUF:2 * NBUF]
        lsem = sems[2 * NBUF]
        c = lax.axis_index("c")
        s = lax.axis_index("s")
        wid = c * 16 + s
        zero16 = jnp.zeros((16,), jnp.float32)

        _load_windows(ei_hbm, pad_hbm, ev, wid, sem=lsem)

        @pl.loop(0, W_EDGE)
        def _(r):
            @pl.loop(0, F, step=16)
            def _(j):
                rows[0, r, pl.ds(j, 16)] = zero16

        nz = ROWS_PER_TILE // W_EDGE
        for kk in range(nz):
            pltpu.async_copy(
                rows.at[0],
                acc_sh.at[pl.ds(s * ROWS_PER_TILE + kk * W_EDGE, W_EDGE)],
                ssem[kk % NBUF])
        for kk in range(nz):
            pltpu.make_async_copy(
                rows.at[0],
                acc_sh.at[pl.ds(s * ROWS_PER_TILE + kk * W_EDGE, W_EDGE)],
                ssem[kk % NBUF]).wait()
        pltpu.make_async_copy(ei_hbm.at[pl.ds(0, PW)], ev, lsem).wait()

        plsc.subcore_barrier()

        def sg(w, i):
            pltpu.async_copy(g_hbm.at[ev.at[w, 0]], rows.at[i], gsem[i])

        def wg(i):
            pltpu.make_async_copy(g_hbm.at[ev.at[0, 0]], rows.at[i], gsem[i]).wait()

        def ss(w, i):
            pltpu.async_copy(rows.at[i], acc_sh.at[ev.at[w, 1]], ssem[i], add=True)

        def ws(i):
            pltpu.make_async_copy(rows.at[i], acc_sh.at[ev.at[0, 1]], ssem[i]).wait()

        for i in range(NBUF):
            sg(i, i)
        for i in range(NBUF):
            wg(i)
            ss(i, i)

        @pl.loop(NBUF, PW, step=NBUF)
        def _(w):
            for i in range(NBUF):
                ws(i)
                sg(w + i, i)
            for i in range(NBUF):
                wg(i)
                ss(w + i, i)

        for i in range(NBUF):
            ws(i)

        plsc.subcore_barrier()

        pltpu.sync_copy(
            acc_sh.at[pl.ds(s * ROWS_PER_TILE, ROWS_PER_TILE)],
            out_hbm.at[c, pl.ds(s * ROWS_PER_TILE, ROWS_PER_TILE)],
        )

    return k


_sc_scatter64 = _make_sc_scatter(64)
_sc_scatter32 = _make_sc_scatter(32)


_BH = 1024
_BQ = 512


def _tc_matmul(xP, W1s):
    def body(x_ref, w_ref, o_ref):
        o_ref[...] = jnp.dot(x_ref[...], w_ref[...],
                             preferred_element_type=jnp.float32)

    return pl.pallas_call(
        body,
        grid=(NH // _BH,),
        in_specs=[
            pl.BlockSpec((_BH, 256), lambda i: (i, 0)),
            pl.BlockSpec((256, 128), lambda i: (0, 0)),
        ],
        out_specs=pl.BlockSpec((_BH, 128), lambda i: (i, 0)),
        out_shape=jax.ShapeDtypeStruct((NH, 128), jnp.float32),
    )(xP, W1s)


def _tc_scale0(degP2, degP4, xwP):
    def body(p2_ref, p4_ref, xw_ref, g_ref, d2_ref, d4_ref):
        a = p2_ref[...]
        d2 = lax.rsqrt(a[0] + a[1] + 1.0)
        b = p4_ref[...]
        d4 = lax.rsqrt(b[0] + b[1] + 1.0)
        d2_ref[...] = d2
        d4_ref[...] = d4
        g_ref[...] = xw_ref[...] * d2

    return pl.pallas_call(
        body,
        grid=(NH // _BH,),
        in_specs=[
            pl.BlockSpec((2, _BH, 128), lambda i: (0, i, 0)),
            pl.BlockSpec((2, _BQ, 128), lambda i: (0, i, 0)),
            pl.BlockSpec((_BH, 128), lambda i: (i, 0)),
        ],
        out_specs=[
            pl.BlockSpec((_BH, 128), lambda i: (i, 0)),
            pl.BlockSpec((_BH, 128), lambda i: (i, 0)),
            pl.BlockSpec((_BQ, 128), lambda i: (i, 0)),
        ],
        out_shape=[
            jax.ShapeDtypeStruct((NH, 128), jnp.float32),
            jax.ShapeDtypeStruct((NH, 128), jnp.float32),
            jax.ShapeDtypeStruct((NQ, 128), jnp.float32),
        ],
    )(degP2, degP4, xwP)


def _tc_layer(partsP, gP, dinvP2, bP, Ws, Kout):
    def body(p_ref, g_ref, d_ref, b_ref, w_ref, o_ref):
        p = p_ref[...]
        d = d_ref[...]
        h = jnp.maximum((p[0] + p[1] + g_ref[...]) * d + b_ref[...], 0.0)
        o_ref[...] = jnp.dot(h * d, w_ref[...],
                             preferred_element_type=jnp.float32)

    return pl.pallas_call(
        body,
        grid=(NH // _BH,),
        in_specs=[
            pl.BlockSpec((2, _BH, 128), lambda i: (0, i, 0)),
            pl.BlockSpec((_BH, 128), lambda i: (i, 0)),
            pl.BlockSpec((_BH, 128), lambda i: (i, 0)),
            pl.BlockSpec((1, 128), lambda i: (0, 0)),
            pl.BlockSpec((128, Kout), lambda i: (0, 0)),
        ],
        out_specs=pl.BlockSpec((_BH, Kout), lambda i: (i, 0)),
        out_shape=jax.ShapeDtypeStruct((NH, Kout), jnp.float32),
    )(partsP, gP, dinvP2, bP.reshape(1, 128), Ws)


def _tc_final(partsQ, gQ, dinvP4, bQ, WcatQ, bcatQ):
    def body(p_ref, g_ref, d_ref, b_ref, w_ref, bc_ref, o_ref):
        p = p_ref[...]
        h = jnp.maximum((p[0] + p[1] + g_ref[...]) * d_ref[...] + b_ref[...],
                        0.0)
        o_ref[...] = jnp.dot(h, w_ref[...],
                             preferred_element_type=jnp.float32) + bc_ref[...]

    return pl.pallas_call(
        body,
        grid=(NQ // _BQ,),
        in_specs=[
            pl.BlockSpec((2, _BQ, 128), lambda i: (0, i, 0)),
            pl.BlockSpec((_BQ, 128), lambda i: (i, 0)),
            pl.BlockSpec((_BQ, 128), lambda i: (i, 0)),
            pl.BlockSpec((1, 128), lambda i: (0, 0)),
            pl.BlockSpec((128, 36), lambda i: (0, 0)),
            pl.BlockSpec((1, 36), lambda i: (0, 0)),
        ],
        out_specs=pl.BlockSpec((_BQ, 36), lambda i: (i, 0)),
        out_shape=jax.ShapeDtypeStruct((N // 4, 36), jnp.float32),
    )(partsQ, gQ, dinvP4, bQ.reshape(1, 128), WcatQ, bcatQ.reshape(1, 36))


def _blockdiag(W, n):
    K, M = W.shape
    out = jnp.zeros((n * K, n * M), W.dtype)
    for i in range(n):
        out = out.at[i * K:(i + 1) * K, i * M:(i + 1) * M].set(W)
    return out


def kernel(x, edge_index, W1, b1, W2, b2, W3, b3, We, be, Wh, bh, Wg, bg):
    ei3 = jnp.transpose(edge_index.reshape(2, RW, W_EDGE), (1, 0, 2))
    xP = jnp.pad(x, ((0, NP - N), (0, 0))).reshape(NH, 256)

    W1s = _blockdiag(W1, 2)
    W2s = _blockdiag(W2, 2)
    W3s = _blockdiag(W3, 2)
    b1P = jnp.concatenate([b1, b1])
    b2P = jnp.concatenate([b2, b2])

    xwP = _tc_matmul(xP, W1s)
    degP2, degP4 = _sc_degree(ei3, _PAD_IDX)
    g1P, d2, d4 = _tc_scale0(degP2, degP4, xwP)

    p1 = _sc_scatter64(g1P.reshape(NP, 64), ei3, _PAD_IDX)
    g2P = _tc_layer(p1.reshape(2, NH, 128), g1P, d2, b1P, W2s, 128)
    p2 = _sc_scatter64(g2P.reshape(NP, 64), ei3, _PAD_IDX)
    g3 = _tc_layer(p2.reshape(2, NH, 128), g2P, d2, b2P, W3s, 64)

    g3d = g3.reshape(NP, 32)
    p3 = _sc_scatter32(g3d, ei3, _PAD_IDX)

    Wcat = jnp.concatenate([We, Wh, Wg], axis=1)
    WcatQ = _blockdiag(Wcat, 4)
    bcat = jnp.concatenate([be, bh, bg])
    b3Q = jnp.concatenate([b3] * 4)
    bcatQ = jnp.concatenate([bcat] * 4)

    out4 = _tc_final(p3.reshape(2, NQ, 128), g3d.reshape(NQ, 128),
                     d4, b3Q, WcatQ, bcatQ)
    return out4.reshape(N, 9)


# speedup vs baseline: 55.4035x; 1.0034x over previous
import functools

import numpy as np

import jax
import jax.numpy as jnp
from jax import lax
from jax.experimental import pallas as pl
from jax.experimental.pallas import tpu as pltpu
from jax.experimental.pallas import tpu_sc as plsc

N = 10000
NP = 10240
E = 320000
W_EDGE = 128
RW = E // W_EDGE
PW = 80
PADW = 32 * PW - RW
N_TILES = 32
ROWS_PER_TILE = NP // 16

_PAD_IDX = np.broadcast_to(
    N + (np.arange(PADW * W_EDGE, dtype=np.int32) % (NP - N)).reshape(PADW, 1, W_EDGE),
    (PADW, 2, W_EDGE),
).copy()

_MESH = plsc.VectorSubcoreMesh(core_axis_name="c", subcore_axis_name="s")

_CP = pltpu.CompilerParams(needs_layout_passes=False)
_CP_UNTILED = pltpu.CompilerParams(use_tc_tiling_on_sc=False)


def _load_windows(ei_hbm, pad_hbm, dest, wid, sem=None):
    cp = pltpu.sync_copy if sem is None else (
        lambda a, b: pltpu.async_copy(a, b, sem))

    @pl.when(wid < N_TILES - 1)
    def _():
        cp(ei_hbm.at[pl.ds(wid * PW, PW)], dest)

    @pl.when(wid == N_TILES - 1)
    def _():
        cp(ei_hbm.at[pl.ds(RW - (PW - PADW), PW - PADW)],
           dest.at[pl.ds(0, PW - PADW)])
        cp(pad_hbm, dest.at[pl.ds(PW - PADW, PADW)])


NH = NP // 2
NQ = NP // 4
HR = NP // 128
HR_T = HR // 16
P2_T = NH // 16
P4_T = NQ // 16


@functools.partial(
    pl.kernel,
    out_type=[
        jax.ShapeDtypeStruct((2, NH, 128), jnp.float32),
        jax.ShapeDtypeStruct((2, NQ, 128), jnp.float32),
    ],
    mesh=_MESH,
    compiler_params=_CP,
    scratch_types=[
        pltpu.VMEM((PW, 2, W_EDGE), jnp.int32),
        pltpu.VMEM((HR, 128), jnp.float32),
        pltpu.VMEM((HR_T, 128), jnp.float32),
        pltpu.VMEM((HR_T, 128), jnp.float32),
        pltpu.VMEM((HR_T, 128), jnp.float32),
        pltpu.VMEM((P2_T, 128), jnp.float32),
        pltpu.VMEM((P4_T, 128), jnp.float32),
        pltpu.VMEM_SHARED((16, HR, 128), jnp.float32),
        pltpu.SemaphoreType.DMA,
        pltpu.SemaphoreType.DMA,
        pltpu.SemaphoreType.DMA,
    ],
)
def _sc_degree(ei_hbm, pad_hbm, p2_hbm, p4_hbm, idx_v, hist_v, hsum, tmp0,
               tmp1, e2, e4, stage, lsem, t0sem, t1sem):
    c = lax.axis_index("c")
    s = lax.axis_index("s")
    wid = c * 16 + s
    zero16 = jnp.zeros((16,), jnp.float32)
    one16 = jnp.ones((16,), jnp.float32)
    tsem = [t0sem, t1sem]
    tbuf = [tmp0, tmp1]

    @pl.loop(0, HR)
    def _(r):
        for j in range(0, 128, 16):
            hist_v[r, pl.ds(j, 16)] = zero16

    _load_windows(ei_hbm, pad_hbm, idx_v, wid)

    @pl.loop(0, PW)
    def _(w):
        for j in range(0, W_EDGE, 16):
            v = idx_v[w, 1, pl.ds(j, 16)]
            plsc.addupdate_scatter(hist_v, [v >> 7, v & 127], one16)

    pltpu.sync_copy(hist_v, stage.at[s])
    plsc.subcore_barrier()

    def stage_cp(j, i):
        pltpu.async_copy(stage.at[j, pl.ds(s * HR_T, HR_T)], tbuf[i], tsem[i])

    def stage_wait(i):
        pltpu.make_async_copy(stage.at[0, pl.ds(s * HR_T, HR_T)], tbuf[i],
                              tsem[i]).wait()

    stage_cp(0, 0)
    stage_cp(1, 1)
    for j in range(16):
        i = j % 2
        stage_wait(i)
        for r in range(HR_T):
            for k in range(8):
                sl = (r, pl.ds(k * 16, 16))
                if j == 0:
                    hsum[sl] = tbuf[i][sl]
                else:
                    hsum[sl] = hsum[sl] + tbuf[i][sl]
        if j + 2 < 16:
            stage_cp(j + 2, i)

    def bcast(val):
        return jnp.full((16,), val, jnp.int32)

    def deg_of(n):
        return plsc.load_gather(hsum, [bcast(n >> 7), bcast(n & 127)])

    @pl.loop(0, P2_T)
    def _(r):
        va = deg_of(2 * r)
        vb = deg_of(2 * r + 1)
        for k in range(4):
            e2[r, pl.ds(k * 16, 16)] = va
        for k in range(4, 8):
            e2[r, pl.ds(k * 16, 16)] = vb

    pltpu.async_copy(e2, p2_hbm.at[c, pl.ds(s * P2_T, P2_T)], lsem)

    @pl.loop(0, P4_T)
    def _(q):
        for i in range(4):
            vv = deg_of(4 * q + i)
            for k in range(2):
                e4[q, pl.ds((2 * i + k) * 16, 16)] = vv

    pltpu.make_async_copy(e2, p2_hbm.at[c, pl.ds(s * P2_T, P2_T)], lsem).wait()
    pltpu.sync_copy(e4, p4_hbm.at[c, pl.ds(s * P4_T, P4_T)])


def _make_sc_scatter(F):
    NBUF = 8

    @functools.partial(
        pl.kernel,
        out_type=jax.ShapeDtypeStruct((2, NP, F), jnp.float32),
        mesh=_MESH,
        compiler_params=_CP_UNTILED,
        scratch_types=[
            pltpu.VMEM((PW, 2, W_EDGE), jnp.int32),
            pltpu.VMEM((NBUF, W_EDGE, F), jnp.float32),
            pltpu.VMEM_SHARED((NP, F), jnp.float32),
        ]
        + [pltpu.SemaphoreType.DMA] * (2 * NBUF + 1),
    )
    def k(g_hbm, ei_hbm, pad_hbm, out_hbm, ev, rows,
          acc_sh, *sems):
        gsem = sems[:NBUF]
        ssem = sems[NBUF:2 * NBUF]
        lsem = sems[2 * NBUF]
        c = lax.axis_index("c")
        s = lax.axis_index("s")
        wid = c * 16 + s
        zero16 = jnp.zeros((16,), jnp.float32)

        _load_windows(ei_hbm, pad_hbm, ev, wid, sem=lsem)

        @pl.loop(0, W_EDGE)
        def _(r):
            @pl.loop(0, F, step=16)
            def _(j):
                rows[0, r, pl.ds(j, 16)] = zero16

        nz = ROWS_PER_TILE // W_EDGE
        for kk in range(nz):
            pltpu.async_copy(
                rows.at[0],
                acc_sh.at[pl.ds(s * ROWS_PER_TILE + kk * W_EDGE, W_EDGE)],
                ssem[kk % NBUF])
        for kk in range(nz):
            pltpu.make_async_copy(
                rows.at[0],
                acc_sh.at[pl.ds(s * ROWS_PER_TILE + kk * W_EDGE, W_EDGE)],
                ssem[kk % NBUF]).wait()
        pltpu.make_async_copy(ei_hbm.at[pl.ds(0, PW)], ev, lsem).wait()

        plsc.subcore_barrier()

        def sg(w, i):
            pltpu.async_copy(g_hbm.at[ev.at[w, 0]], rows.at[i], gsem[i])

        def wg(i):
            pltpu.make_async_copy(g_hbm.at[ev.at[0, 0]], rows.at[i], gsem[i]).wait()

        def ss(w, i):
            pltpu.async_copy(rows.at[i], acc_sh.at[ev.at[w, 1]], ssem[i], add=True)

        def ws(i):
            pltpu.make_async_copy(rows.at[i], acc_sh.at[ev.at[0, 1]], ssem[i]).wait()

        for i in range(NBUF):
            sg(i, i)
        for i in range(NBUF):
            wg(i)
            ss(i, i)

        @pl.loop(NBUF, PW, step=NBUF)
        def _(w):
            for i in range(NBUF):
                ws(i)
                sg(w + i, i)
            for i in range(NBUF):
                wg(i)
                ss(w + i, i)

        for i in range(NBUF):
            ws(i)

        plsc.subcore_barrier()

        pltpu.sync_copy(
            acc_sh.at[pl.ds(s * ROWS_PER_TILE, ROWS_PER_TILE)],
            out_hbm.at[c, pl.ds(s * ROWS_PER_TILE, ROWS_PER_TILE)],
        )

    return k


_sc_scatter64 = _make_sc_scatter(64)
_sc_scatter32 = _make_sc_scatter(32)


_BH = 1024
_BQ = 512


def _tc_matmul(xP, W1s):
    def body(x_ref, w_ref, o_ref):
        o_ref[...] = jnp.dot(x_ref[...], w_ref[...],
                             preferred_element_type=jnp.float32)

    return pl.pallas_call(
        body,
        grid=(NH // _BH,),
        in_specs=[
            pl.BlockSpec((_BH, 256), lambda i: (i, 0)),
            pl.BlockSpec((256, 128), lambda i: (0, 0)),
        ],
        out_specs=pl.BlockSpec((_BH, 128), lambda i: (i, 0)),
        out_shape=jax.ShapeDtypeStruct((NH, 128), jnp.float32),
    )(xP, W1s)


def _tc_scale0(degP2, xwP):
    def body(p2_ref, xw_ref, g_ref, d2_ref):
        a = p2_ref[...]
        d2 = lax.rsqrt(a[0] + a[1] + 1.0)
        d2_ref[...] = d2
        g_ref[...] = xw_ref[...] * d2

    return pl.pallas_call(
        body,
        grid=(NH // _BH,),
        in_specs=[
            pl.BlockSpec((2, _BH, 128), lambda i: (0, i, 0)),
            pl.BlockSpec((_BH, 128), lambda i: (i, 0)),
        ],
        out_specs=[
            pl.BlockSpec((_BH, 128), lambda i: (i, 0)),
            pl.BlockSpec((_BH, 128), lambda i: (i, 0)),
        ],
        out_shape=[
            jax.ShapeDtypeStruct((NH, 128), jnp.float32),
            jax.ShapeDtypeStruct((NH, 128), jnp.float32),
        ],
    )(degP2, xwP)


def _tc_layer(partsP, gP, dinvP2, bP, Ws, Kout):
    def body(p_ref, g_ref, d_ref, b_ref, w_ref, o_ref):
        p = p_ref[...]
        d = d_ref[...]
        h = jnp.maximum((p[0] + p[1] + g_ref[...]) * d + b_ref[...], 0.0)
        o_ref[...] = jnp.dot(h * d, w_ref[...],
                             preferred_element_type=jnp.float32)

    return pl.pallas_call(
        body,
        grid=(NH // _BH,),
        in_specs=[
            pl.BlockSpec((2, _BH, 128), lambda i: (0, i, 0)),
            pl.BlockSpec((_BH, 128), lambda i: (i, 0)),
            pl.BlockSpec((_BH, 128), lambda i: (i, 0)),
            pl.BlockSpec((1, 128), lambda i: (0, 0)),
            pl.BlockSpec((128, Kout), lambda i: (0, 0)),
        ],
        out_specs=pl.BlockSpec((_BH, Kout), lambda i: (i, 0)),
        out_shape=jax.ShapeDtypeStruct((NH, Kout), jnp.float32),
    )(partsP, gP, dinvP2, bP.reshape(1, 128), Ws)


def _tc_final(partsQ, gQ, degP4, bQ, WcatQ, bcatQ):
    def body(p_ref, g_ref, d_ref, b_ref, w_ref, bc_ref, o_ref):
        p = p_ref[...]
        dq = d_ref[...]
        d4 = lax.rsqrt(dq[0] + dq[1] + 1.0)
        h = jnp.maximum((p[0] + p[1] + g_ref[...]) * d4 + b_ref[...],
                        0.0)
        o_ref[...] = jnp.dot(h, w_ref[...],
                             preferred_element_type=jnp.float32) + bc_ref[...]

    return pl.pallas_call(
        body,
        grid=(NQ // _BQ,),
        in_specs=[
            pl.BlockSpec((2, _BQ, 128), lambda i: (0, i, 0)),
            pl.BlockSpec((_BQ, 128), lambda i: (i, 0)),
            pl.BlockSpec((2, _BQ, 128), lambda i: (0, i, 0)),
            pl.BlockSpec((1, 128), lambda i: (0, 0)),
            pl.BlockSpec((128, 36), lambda i: (0, 0)),
            pl.BlockSpec((1, 36), lambda i: (0, 0)),
        ],
        out_specs=pl.BlockSpec((_BQ, 36), lambda i: (i, 0)),
        out_shape=jax.ShapeDtypeStruct((N // 4, 36), jnp.float32),
    )(partsQ, gQ, degP4, bQ.reshape(1, 128), WcatQ, bcatQ.reshape(1, 36))


def _blockdiag(W, n):
    K, M = W.shape
    out = jnp.zeros((n * K, n * M), W.dtype)
    for i in range(n):
        out = out.at[i * K:(i + 1) * K, i * M:(i + 1) * M].set(W)
    return out


def kernel(x, edge_index, W1, b1, W2, b2, W3, b3, We, be, Wh, bh, Wg, bg):
    ei3 = jnp.transpose(edge_index.reshape(2, RW, W_EDGE), (1, 0, 2))
    xP = jnp.pad(x, ((0, NP - N), (0, 0))).reshape(NH, 256)

    W1s = _blockdiag(W1, 2)
    W2s = _blockdiag(W2, 2)
    W3s = _blockdiag(W3, 2)
    b1P = jnp.concatenate([b1, b1])
    b2P = jnp.concatenate([b2, b2])

    xwP = _tc_matmul(xP, W1s)
    degP2, degP4 = _sc_degree(ei3, _PAD_IDX)
    g1P, d2 = _tc_scale0(degP2, xwP)

    p1 = _sc_scatter64(g1P.reshape(NP, 64), ei3, _PAD_IDX)
    g2P = _tc_layer(p1.reshape(2, NH, 128), g1P, d2, b1P, W2s, 128)
    p2 = _sc_scatter64(g2P.reshape(NP, 64), ei3, _PAD_IDX)
    g3 = _tc_layer(p2.reshape(2, NH, 128), g2P, d2, b2P, W3s, 64)

    g3d = g3.reshape(NP, 32)
    p3 = _sc_scatter32(g3d, ei3, _PAD_IDX)

    Wcat = jnp.concatenate([We, Wh, Wg], axis=1)
    WcatQ = _blockdiag(Wcat, 4)
    bcat = jnp.concatenate([be, bh, bg])
    b3Q = jnp.concatenate([b3] * 4)
    bcatQ = jnp.concatenate([bcat] * 4)

    out4 = _tc_final(p3.reshape(2, NQ, 128), g3d.reshape(NQ, 128),
                     degP4, b3Q, WcatQ, bcatQ)
    return out4.reshape(N, 9)


# speedup vs baseline: 56.3761x; 1.0176x over previous
import functools

import numpy as np

import jax
import jax.numpy as jnp
from jax import lax
from jax.experimental import pallas as pl
from jax.experimental.pallas import tpu as pltpu
from jax.experimental.pallas import tpu_sc as plsc

N = 10000
NP = 10240
E = 320000
W_EDGE = 128
RW = E // W_EDGE
PW = 80
PADW = 32 * PW - RW
N_TILES = 32
ROWS_PER_TILE = NP // 16

_PAD_IDX = np.broadcast_to(
    N + (np.arange(PADW * W_EDGE, dtype=np.int32) % (NP - N)).reshape(PADW, 1, W_EDGE),
    (PADW, 2, W_EDGE),
).copy()

_MESH = plsc.VectorSubcoreMesh(core_axis_name="c", subcore_axis_name="s")

_CP = pltpu.CompilerParams(needs_layout_passes=False)
_CP_UNTILED = pltpu.CompilerParams(use_tc_tiling_on_sc=False)


def _load_windows(ei_hbm, pad_hbm, dest, wid, sem=None):
    cp = pltpu.sync_copy if sem is None else (
        lambda a, b: pltpu.async_copy(a, b, sem))

    @pl.when(wid < N_TILES - 1)
    def _():
        cp(ei_hbm.at[pl.ds(wid * PW, PW)], dest)

    @pl.when(wid == N_TILES - 1)
    def _():
        cp(ei_hbm.at[pl.ds(RW - (PW - PADW), PW - PADW)],
           dest.at[pl.ds(0, PW - PADW)])
        cp(pad_hbm, dest.at[pl.ds(PW - PADW, PADW)])


NH = NP // 2
NQ = NP // 4
HR = NP // 128
HR_T = HR // 16
P2_T = NH // 16
P4_T = NQ // 16


@functools.partial(
    pl.kernel,
    out_type=[
        jax.ShapeDtypeStruct((2, NH, 128), jnp.float32),
        jax.ShapeDtypeStruct((2, NQ, 128), jnp.float32),
    ],
    mesh=_MESH,
    compiler_params=_CP,
    scratch_types=[
        pltpu.VMEM((PW, 2, W_EDGE), jnp.int32),
        pltpu.VMEM((HR, 128), jnp.float32),
        pltpu.VMEM((HR_T, 128), jnp.float32),
        pltpu.VMEM((HR_T, 128), jnp.float32),
        pltpu.VMEM((HR_T, 128), jnp.float32),
        pltpu.VMEM((P2_T, 128), jnp.float32),
        pltpu.VMEM((P4_T, 128), jnp.float32),
        pltpu.VMEM_SHARED((16, HR, 128), jnp.float32),
        pltpu.SemaphoreType.DMA,
        pltpu.SemaphoreType.DMA,
        pltpu.SemaphoreType.DMA,
    ],
)
def _sc_degree(ei_hbm, pad_hbm, p2_hbm, p4_hbm, idx_v, hist_v, hsum, tmp0,
               tmp1, e2, e4, stage, lsem, t0sem, t1sem):
    c = lax.axis_index("c")
    s = lax.axis_index("s")
    wid = c * 16 + s
    zero16 = jnp.zeros((16,), jnp.float32)
    one16 = jnp.ones((16,), jnp.float32)
    tsem = [t0sem, t1sem]
    tbuf = [tmp0, tmp1]

    @pl.loop(0, HR)
    def _(r):
        for j in range(0, 128, 16):
            hist_v[r, pl.ds(j, 16)] = zero16

    _load_windows(ei_hbm, pad_hbm, idx_v, wid)

    @pl.loop(0, PW)
    def _(w):
        for j in range(0, W_EDGE, 16):
            v = idx_v[w, 1, pl.ds(j, 16)]
            plsc.addupdate_scatter(hist_v, [v >> 7, v & 127], one16)

    pltpu.sync_copy(hist_v, stage.at[s])
    plsc.subcore_barrier()

    def stage_cp(j, i):
        pltpu.async_copy(stage.at[j, pl.ds(s * HR_T, HR_T)], tbuf[i], tsem[i])

    def stage_wait(i):
        pltpu.make_async_copy(stage.at[0, pl.ds(s * HR_T, HR_T)], tbuf[i],
                              tsem[i]).wait()

    stage_cp(0, 0)
    stage_cp(1, 1)
    for j in range(16):
        i = j % 2
        stage_wait(i)
        for r in range(HR_T):
            for k in range(8):
                sl = (r, pl.ds(k * 16, 16))
                if j == 0:
                    hsum[sl] = tbuf[i][sl]
                else:
                    hsum[sl] = hsum[sl] + tbuf[i][sl]
        if j + 2 < 16:
            stage_cp(j + 2, i)

    def bcast(val):
        return jnp.full((16,), val, jnp.int32)

    def deg_of(n):
        return plsc.load_gather(hsum, [bcast(n >> 7), bcast(n & 127)])

    @pl.loop(0, P2_T)
    def _(r):
        va = deg_of(2 * r)
        vb = deg_of(2 * r + 1)
        for k in range(4):
            e2[r, pl.ds(k * 16, 16)] = va
        for k in range(4, 8):
            e2[r, pl.ds(k * 16, 16)] = vb

    pltpu.async_copy(e2, p2_hbm.at[c, pl.ds(s * P2_T, P2_T)], lsem)

    @pl.loop(0, P4_T)
    def _(q):
        for i in range(4):
            vv = deg_of(4 * q + i)
            for k in range(2):
                e4[q, pl.ds((2 * i + k) * 16, 16)] = vv

    pltpu.make_async_copy(e2, p2_hbm.at[c, pl.ds(s * P2_T, P2_T)], lsem).wait()
    pltpu.sync_copy(e4, p4_hbm.at[c, pl.ds(s * P4_T, P4_T)])


def _make_sc_scatter(F):
    NBUF = 8

    @functools.partial(
        pl.kernel,
        out_type=jax.ShapeDtypeStruct((2, NP, F), jnp.float32),
        mesh=_MESH,
        compiler_params=_CP_UNTILED,
        scratch_types=[
            pltpu.VMEM((PW, 2, W_EDGE), jnp.int32),
            pltpu.VMEM((NBUF, W_EDGE, F), jnp.float32),
            pltpu.VMEM_SHARED((NP, F), jnp.float32),
        ]
        + [pltpu.SemaphoreType.DMA] * (2 * NBUF + 1),
    )
    def k(g_hbm, ei_hbm, pad_hbm, out_hbm, ev, rows,
          acc_sh, *sems):
        gsem = sems[:NBUF]
        ssem = sems[NBUF:2 * NBUF]
        lsem = sems[2 * NBUF]
        c = lax.axis_index("c")
        s = lax.axis_index("s")
        wid = c * 16 + s
        zero16 = jnp.zeros((16,), jnp.float32)

        _load_windows(ei_hbm, pad_hbm, ev, wid, sem=lsem)

        @pl.loop(0, W_EDGE)
        def _(r):
            @pl.loop(0, F, step=16)
            def _(j):
                rows[0, r, pl.ds(j, 16)] = zero16

        nz = ROWS_PER_TILE // W_EDGE
        for kk in range(nz):
            pltpu.async_copy(
                rows.at[0],
                acc_sh.at[pl.ds(s * ROWS_PER_TILE + kk * W_EDGE, W_EDGE)],
                ssem[kk % NBUF])
        for kk in range(nz):
            pltpu.make_async_copy(
                rows.at[0],
                acc_sh.at[pl.ds(s * ROWS_PER_TILE + kk * W_EDGE, W_EDGE)],
                ssem[kk % NBUF]).wait()
        pltpu.make_async_copy(ei_hbm.at[pl.ds(0, PW)], ev, lsem).wait()

        plsc.subcore_barrier()

        def sg(w, i):
            pltpu.async_copy(g_hbm.at[ev.at[w, 0]], rows.at[i], gsem[i])

        def wg(i):
            pltpu.make_async_copy(g_hbm.at[ev.at[0, 0]], rows.at[i], gsem[i]).wait()

        def ss(w, i):
            pltpu.async_copy(rows.at[i], acc_sh.at[ev.at[w, 1]], ssem[i], add=True)

        def ws(i):
            pltpu.make_async_copy(rows.at[i], acc_sh.at[ev.at[0, 1]], ssem[i]).wait()

        for i in range(NBUF):
            sg(i, i)
        for i in range(NBUF):
            wg(i)
            ss(i, i)

        @pl.loop(NBUF, PW, step=NBUF)
        def _(w):
            for i in range(NBUF):
                ws(i)
                sg(w + i, i)
            for i in range(NBUF):
                wg(i)
                ss(w + i, i)

        for i in range(NBUF):
            ws(i)

        plsc.subcore_barrier()

        pltpu.sync_copy(
            acc_sh.at[pl.ds(s * ROWS_PER_TILE, ROWS_PER_TILE)],
            out_hbm.at[c, pl.ds(s * ROWS_PER_TILE, ROWS_PER_TILE)],
        )

    return k


_sc_scatter64 = _make_sc_scatter(64)
_sc_scatter32 = _make_sc_scatter(32)


_BH = 1280
_BQ = 640


def _tc_matmul(xP, W1s):
    def body(x_ref, w_ref, o_ref):
        o_ref[...] = jnp.dot(x_ref[...], w_ref[...],
                             preferred_element_type=jnp.float32)

    return pl.pallas_call(
        body,
        grid=(NH // _BH,),
        in_specs=[
            pl.BlockSpec((_BH, 256), lambda i: (i, 0)),
            pl.BlockSpec((256, 128), lambda i: (0, 0)),
        ],
        out_specs=pl.BlockSpec((_BH, 128), lambda i: (i, 0)),
        out_shape=jax.ShapeDtypeStruct((NH, 128), jnp.float32),
    )(xP, W1s)


def _tc_scale0(degP2, xwP):
    def body(p2_ref, xw_ref, g_ref, d2_ref):
        a = p2_ref[...]
        d2 = lax.rsqrt(a[0] + a[1] + 1.0)
        d2_ref[...] = d2
        g_ref[...] = xw_ref[...] * d2

    return pl.pallas_call(
        body,
        grid=(NH // _BH,),
        in_specs=[
            pl.BlockSpec((2, _BH, 128), lambda i: (0, i, 0)),
            pl.BlockSpec((_BH, 128), lambda i: (i, 0)),
        ],
        out_specs=[
            pl.BlockSpec((_BH, 128), lambda i: (i, 0)),
            pl.BlockSpec((_BH, 128), lambda i: (i, 0)),
        ],
        out_shape=[
            jax.ShapeDtypeStruct((NH, 128), jnp.float32),
            jax.ShapeDtypeStruct((NH, 128), jnp.float32),
        ],
    )(degP2, xwP)


def _tc_layer(partsP, gP, dinvP2, bP, Ws, Kout):
    def body(p_ref, g_ref, d_ref, b_ref, w_ref, o_ref):
        p = p_ref[...]
        d = d_ref[...]
        h = jnp.maximum((p[0] + p[1] + g_ref[...]) * d + b_ref[...], 0.0)
        o_ref[...] = jnp.dot(h * d, w_ref[...],
                             preferred_element_type=jnp.float32)

    return pl.pallas_call(
        body,
        grid=(NH // _BH,),
        in_specs=[
            pl.BlockSpec((2, _BH, 128), lambda i: (0, i, 0)),
            pl.BlockSpec((_BH, 128), lambda i: (i, 0)),
            pl.BlockSpec((_BH, 128), lambda i: (i, 0)),
            pl.BlockSpec((1, 128), lambda i: (0, 0)),
            pl.BlockSpec((128, Kout), lambda i: (0, 0)),
        ],
        out_specs=pl.BlockSpec((_BH, Kout), lambda i: (i, 0)),
        out_shape=jax.ShapeDtypeStruct((NH, Kout), jnp.float32),
    )(partsP, gP, dinvP2, bP.reshape(1, 128), Ws)


def _tc_final(partsQ, gQ, degP4, bQ, WcatQ, bcatQ):
    def body(p_ref, g_ref, d_ref, b_ref, w_ref, bc_ref, o_ref):
        p = p_ref[...]
        dq = d_ref[...]
        d4 = lax.rsqrt(dq[0] + dq[1] + 1.0)
        h = jnp.maximum((p[0] + p[1] + g_ref[...]) * d4 + b_ref[...],
                        0.0)
        o_ref[...] = jnp.dot(h, w_ref[...],
                             preferred_element_type=jnp.float32) + bc_ref[...]

    return pl.pallas_call(
        body,
        grid=(NQ // _BQ,),
        in_specs=[
            pl.BlockSpec((2, _BQ, 128), lambda i: (0, i, 0)),
            pl.BlockSpec((_BQ, 128), lambda i: (i, 0)),
            pl.BlockSpec((2, _BQ, 128), lambda i: (0, i, 0)),
            pl.BlockSpec((1, 128), lambda i: (0, 0)),
            pl.BlockSpec((128, 36), lambda i: (0, 0)),
            pl.BlockSpec((1, 36), lambda i: (0, 0)),
        ],
        out_specs=pl.BlockSpec((_BQ, 36), lambda i: (i, 0)),
        out_shape=jax.ShapeDtypeStruct((N // 4, 36), jnp.float32),
    )(partsQ, gQ, degP4, bQ.reshape(1, 128), WcatQ, bcatQ.reshape(1, 36))


def _blockdiag(W, n):
    K, M = W.shape
    out = jnp.zeros((n * K, n * M), W.dtype)
    for i in range(n):
        out = out.at[i * K:(i + 1) * K, i * M:(i + 1) * M].set(W)
    return out


def kernel(x, edge_index, W1, b1, W2, b2, W3, b3, We, be, Wh, bh, Wg, bg):
    ei3 = jnp.transpose(edge_index.reshape(2, RW, W_EDGE), (1, 0, 2))
    xP = jnp.pad(x, ((0, NP - N), (0, 0))).reshape(NH, 256)

    W1s = _blockdiag(W1, 2)
    W2s = _blockdiag(W2, 2)
    W3s = _blockdiag(W3, 2)
    b1P = jnp.concatenate([b1, b1])
    b2P = jnp.concatenate([b2, b2])

    xwP = _tc_matmul(xP, W1s)
    degP2, degP4 = _sc_degree(ei3, _PAD_IDX)
    g1P, d2 = _tc_scale0(degP2, xwP)

    p1 = _sc_scatter64(g1P.reshape(NP, 64), ei3, _PAD_IDX)
    g2P = _tc_layer(p1.reshape(2, NH, 128), g1P, d2, b1P, W2s, 128)
    p2 = _sc_scatter64(g2P.reshape(NP, 64), ei3, _PAD_IDX)
    g3 = _tc_layer(p2.reshape(2, NH, 128), g2P, d2, b2P, W3s, 64)

    g3d = g3.reshape(NP, 32)
    p3 = _sc_scatter32(g3d, ei3, _PAD_IDX)

    Wcat = jnp.concatenate([We, Wh, Wg], axis=1)
    WcatQ = _blockdiag(Wcat, 4)
    bcat = jnp.concatenate([be, bh, bg])
    b3Q = jnp.concatenate([b3] * 4)
    bcatQ = jnp.concatenate([bcat] * 4)

    out4 = _tc_final(p3.reshape(2, NQ, 128), g3d.reshape(NQ, 128),
                     degP4, b3Q, WcatQ, bcatQ)
    return out4.reshape(N, 9)
